# baseline (speedup 1.0000x reference)
import functools

import jax
import jax.numpy as jnp
from jax import lax
from jax.experimental import pallas as pl
from jax.experimental.pallas import tpu as pltpu
from jax.experimental.pallas import tpu_sc as plsc

_B = 4096
_D = 16
_NU = 23
_NC = 3
_NI = 8
_HL = 200
_VU = 100000
_VC = 1000
_VI0 = 1000000
_VIR = 100000
_UNUM = 4
_INUM = 6
_HID = 128
_TOW = 64
_UIN = _NU * _D + _NC * _D + 2 * _D
_IIN = _NI * _D + _D

_NW = 32
_RPT = _B // _NW
_RPS = 4
_NST = _RPT // _NW * _NW // _RPS * 1
_NST = _RPT // _RPS
_SID = _RPS * _HL

_CW = 1024
_NCH = _VU // _CW
_TW = _VU - _NCH * _CW


def _repack_body(it0, utT, irT, it0L, utL, irL,
                 srcv, colvA, colvB, tailv, stg, stg2):
    info = plsc.get_sparse_core_info()
    wid = lax.axis_index("s") * info.num_cores + lax.axis_index("c")

    nfull = _VI0 // 128
    npt = nfull // _NW + 1

    def it0_chunk(t, carry):
        c = wid + t * _NW

        @pl.when(c < nfull)
        def _():
            r0 = pl.multiple_of(c * 128, 8)
            pltpu.sync_copy(it0.at[pl.ds(r0, 128), :], srcv)

            def rbody(r, cc):
                stg[pl.ds(pl.multiple_of(r * 16, 16), 16)] = srcv[r, :]
                return cc
            lax.fori_loop(0, 128, rbody, 0)
            pltpu.sync_copy(stg, it0L.at[pl.ds(c * 2048, 2048)])
        return carry
    lax.fori_loop(0, npt, it0_chunk, 0)

    @pl.when(wid == 0)
    def _():
        r0 = nfull * 128
        pltpu.sync_copy(it0.at[pl.ds(r0, 64), :], srcv.at[pl.ds(0, 64), :])

        def rbody(r, cc):
            stg[pl.ds(pl.multiple_of(r * 16, 16), 16)] = srcv[r, :]
            return cc
        lax.fori_loop(0, 64, rbody, 0)
        pltpu.sync_copy(stg.at[pl.ds(0, 1024)],
                        it0L.at[pl.ds(r0 * 16, 1024)])

    def do_chunk(src3, f, i0, cw, buf, out1d, v):
        nc = (cw + 15) // 16
        pltpu.sync_copy(src3.at[f, pl.ds(0, 8), pl.ds(i0, cw)],
                        buf.at[pl.ds(0, 8), pl.ds(0, cw)])
        pltpu.sync_copy(src3.at[f, pl.ds(8, 8), pl.ds(i0, cw)],
                        buf.at[pl.ds(8, 8), pl.ds(0, cw)])

        def jrow(j, carry):
            def cbody(c, cc):
                o = pl.multiple_of(lax.min(c * 16, cw - 16), 8)
                stg2[pl.ds(pl.multiple_of(j * cw + o, 8), 16)] = \
                    buf[j, pl.ds(o, 16)]
                return cc
            lax.fori_loop(0, nc, cbody, 0)
            return carry
        lax.fori_loop(0, 16, jrow, 0)

        def jout(j, carry):
            pltpu.sync_copy(stg2.at[pl.ds(pl.multiple_of(j * cw, 8), cw)],
                            out1d.at[pl.ds(pl.multiple_of(
                                (f * 16 + j) * v + i0, 8), cw)])
            return carry
        lax.fori_loop(0, 16, jout, 0)

    nut = _NU * _NCH

    def ut_chunk(t, carry):
        c = wid + t * _NW

        @pl.when(c < nut)
        def _():
            f = c // _NCH
            i0 = pl.multiple_of((c - f * _NCH) * _CW, 128)
            do_chunk(utT, f, i0, _CW, colvA, utL, _VU)
        return carry
    lax.fori_loop(0, nut // _NW + 1, ut_chunk, 0)

    nir = (_NI - 1) * _NCH

    def ir_chunk(t, carry):
        c = wid + t * _NW

        @pl.when(c < nir)
        def _():
            f = c // _NCH
            i0 = pl.multiple_of((c - f * _NCH) * _CW, 128)
            do_chunk(irT, f, i0, _CW, colvB, irL, _VIR)
        return carry
    lax.fori_loop(0, nir // _NW + 1, ir_chunk, 0)

    @pl.when(wid < _NU)
    def _():
        do_chunk(utT, wid, _NCH * _CW, _TW, tailv, utL, _VU)

    @pl.when(jnp.logical_and(wid >= _NU, wid < _NU + _NI - 1))
    def _():
        do_chunk(irT, wid - _NU, _NCH * _CW, _TW, tailv, irL, _VIR)


def _sc_body(ucat, ccat, icat, histf, hmaskf, unum, inum, wun, bun2, win,
             bin2, utf, ctf, it0, irf, x_out, y_out,
             idxA, idxB, idxTA, idxTB, colgA, colgB, embA, embB,
             ids_all, hmask_all, hrA, hrB,
             pool_v, num_v, un_v, in_v, wun_v, bun_v, win_v, bin_v,
             semA, semB):
    info = plsc.get_sparse_core_info()
    wid = lax.axis_index("s") * info.num_cores + lax.axis_index("c")
    b0 = wid * _RPT
    bs = pl.ds(b0, _RPT)

    pltpu.sync_copy(wun, wun_v)
    pltpu.sync_copy(bun2, bun_v)
    pltpu.sync_copy(win, win_v)
    pltpu.sync_copy(bin2, bin_v)
    pltpu.sync_copy(unum.at[pl.ds(b0 * _D, _RPT * _D)], un_v)
    pltpu.sync_copy(inum.at[pl.ds(b0 * _D, _RPT * _D)], in_v)

    def unum_body(r, carry):
        uvec = un_v[pl.ds(pl.multiple_of(r * _D, _D), _D)]
        acc = bun_v[...]
        for k in range(_UNUM):
            acc = acc + uvec[k] * wun_v[pl.ds(k * _D, _D)]
        num_v[r, :] = acc
        return carry
    lax.fori_loop(0, _RPT, unum_body, 0)
    pltpu.sync_copy(num_v, x_out.at[bs, pl.ds(26 * _D, _D)])

    def inum_body(r, carry):
        ivec = in_v[pl.ds(pl.multiple_of(r * _D, _D), _D)]
        acc = bin_v[...]
        for k in range(_INUM):
            acc = acc + ivec[k] * win_v[pl.ds(k * _D, _D)]
        num_v[r, :] = acc
        return carry
    lax.fori_loop(0, _RPT, inum_body, 0)
    pltpu.sync_copy(num_v, y_out.at[bs, pl.ds(_NI * _D, _D)])

    i16 = lax.iota(jnp.int32, 16)

    def wprep(srcarr, f, v, table, idx_v, idxT_v, colg_v, sem):
        pltpu.sync_copy(
            srcarr.at[pl.ds(pl.multiple_of(f * _B + b0, 8), _RPT)], idx_v)
        jv = i16 * v + (f * 16) * v

        def rxf(c, carry):
            idv = idx_v[pl.ds(pl.multiple_of(c * 16, 16), 16)]
            for m in range(16):
                r = c * 16 + m
                idxT_v[pl.ds(pl.multiple_of(r * 16, 16), 16)] = \
                    jv + idv[m]
            return carry
        lax.fori_loop(0, _RPT // 16, rxf, 0)

        def jissue(j, carry):
            jb = pl.multiple_of(j * 128, 128)
            pltpu.async_copy(table.at[idxT_v.at[pl.ds(jb, 128)]],
                             colg_v.at[pl.ds(jb, 128)], sem)
            return carry
        lax.fori_loop(0, 16, jissue, 0)

    def wfinish(dstbuf, col, colg_v, emb_v, sem):
        def jdrain(j, carry):
            jb = pl.multiple_of(j * 128, 128)
            pltpu.make_async_copy(utf.at[pl.ds(0, 128)],
                                  colg_v.at[pl.ds(jb, 128)], sem).wait()
            return carry
        lax.fori_loop(0, 16, jdrain, 0)

        def rtrans(rb, carry):
            for m in range(16):
                r = rb * 16 + m
                emb_v[r, :] = colg_v[pl.ds(pl.multiple_of(r * 16, 16), 16)]
            return carry
        lax.fori_loop(0, _RPT // 16, rtrans, 0)
        pltpu.sync_copy(emb_v,
                        dstbuf.at[bs, pl.ds(pl.multiple_of(col, 16), _D)])

    def ugrp(t, carry):
        f1 = 2 * t
        f2 = 2 * t + 1
        wprep(ucat, f1, _VU, utf, idxA, idxTA, colgA, semA)

        @pl.when(f2 < _NU)
        def _():
            wprep(ucat, f2, _VU, utf, idxB, idxTB, colgB, semB)
        wfinish(x_out, f1 * _D, colgA, embA, semA)

        @pl.when(f2 < _NU)
        def _():
            wfinish(x_out, f2 * _D, colgB, embB, semB)
        return carry
    lax.fori_loop(0, (_NU + 1) // 2, ugrp, 0)

    def irun(f, idx_v, idxT_v, colg_v, emb_v, sem):
        pltpu.sync_copy(
            icat.at[pl.ds(pl.multiple_of((f + 1) * _B + b0, 8), _RPT)],
            idx_v)
        jv = i16 * _VIR + (f * 16) * _VIR

        def rxf(c, carry):
            idv = idx_v[pl.ds(pl.multiple_of(c * 16, 16), 16)]
            for m in range(16):
                r = c * 16 + m
                idxT_v[pl.ds(pl.multiple_of(r * 16, 16), 16)] = \
                    jv + idv[m]
            return carry
        lax.fori_loop(0, _RPT // 16, rxf, 0)

        def jissue(j, carry):
            jb = pl.multiple_of(j * 128, 128)
            pltpu.async_copy(irf.at[idxT_v.at[pl.ds(jb, 128)]],
                             colg_v.at[pl.ds(jb, 128)], sem)
            return carry
        lax.fori_loop(0, 16, jissue, 0)

    def igrp2(t, carry):
        f1 = 2 * t
        f2 = 2 * t + 1
        irun(f1, idxA, idxTA, colgA, embA, semA)

        @pl.when(f2 < _NI - 1)
        def _():
            irun(f2, idxB, idxTB, colgB, embB, semB)
        wfinish(y_out, (f1 + 1) * _D, colgA, embA, semA)

        @pl.when(f2 < _NI - 1)
        def _():
            wfinish(y_out, (f2 + 1) * _D, colgB, embB, semB)
        return carry
    lax.fori_loop(0, _NI // 2, igrp2, 0)

    for f in range(_NC):
        pltpu.sync_copy(ccat.at[pl.ds(f * _B + b0, _RPT)], idxA)
        if f:
            for c in range(_RPT // 16):
                sl = pl.ds(c * 16, 16)
                idxA[sl] = idxA[sl] + f * _VC
        pltpu.async_copy(ctf.at[idxA], embA, semA).wait()
        pltpu.sync_copy(embA, x_out.at[bs, pl.ds((_NU + f) * _D, _D)])

    pltpu.sync_copy(icat.at[pl.ds(b0, _RPT)], idxA)
    pltpu.async_copy(it0.at[idxA], embA, semA).wait()
    pltpu.sync_copy(embA, y_out.at[bs, pl.ds(0, _D)])

    pltpu.sync_copy(histf.at[pl.ds(b0 * _HL, _RPT * _HL)], ids_all)
    pltpu.sync_copy(hmaskf.at[pl.ds(b0 * _HL, _RPT * _HL)], hmask_all)

    def issue_stage(s, buf, sem):
        for k in range(_RPS):
            o = pl.multiple_of(s * _SID + k * _HL, 8)
            pltpu.async_copy(it0.at[ids_all.at[pl.ds(o, 128)]],
                             buf.at[pl.ds(k * _HL, 128)], sem)
            pltpu.async_copy(it0.at[ids_all.at[pl.ds(o + 128, _HL - 128)]],
                             buf.at[pl.ds(k * _HL + 128, _HL - 128)], sem)

    def drain_stage(buf, sem):
        for k in range(_RPS):
            pltpu.make_async_copy(it0.at[pl.ds(0, 128)],
                                  buf.at[pl.ds(k * _HL, 128)], sem).wait()
            pltpu.make_async_copy(it0.at[pl.ds(0, _HL - 128)],
                                  buf.at[pl.ds(k * _HL + 128, _HL - 128)],
                                  sem).wait()

    def compute_stage(s, buf):
        for k in range(_RPS):
            mbase = s * _SID + k * _HL
            zv = jnp.zeros((16,), jnp.float32)

            def acc_body(c, carry2):
                accs, ms = carry2
                accs = list(accs)
                mvec = hmask_all[pl.ds(pl.multiple_of(mbase + c * 16, 8), 16)]
                base = k * _HL + c * 16
                for j in range(16):
                    mj = mvec[j]
                    accs[j % 4] = accs[j % 4] + buf[base + j, :] * mj
                    ms = ms + mj
                return (tuple(accs), ms)
            accs, ms = lax.fori_loop(
                0, 12, acc_body, ((zv, zv, zv, zv), jnp.float32(0.0)))
            a0, a1, a2, a3 = accs
            mvec = hmask_all[pl.ds(pl.multiple_of(mbase + 192, 8), 16)]
            for j in range(8):
                mj = mvec[j]
                a0 = a0 + buf[k * _HL + 192 + j, :] * mj
                ms = ms + mj
            a = (a0 + a1) + (a2 + a3)
            pool_v[s * _RPS + k, :] = a / jnp.maximum(ms, 1e-6)

    issue_stage(0, hrA, semA)

    def hist_loop(t, carry):
        sA = 2 * t
        sB = 2 * t + 1
        issue_stage(sB, hrB, semB)
        drain_stage(hrA, semA)
        compute_stage(sA, hrA)
        issue_stage(lax.rem(sA + 2, _NST), hrA, semA)
        drain_stage(hrB, semB)
        compute_stage(sB, hrB)
        return carry
    lax.fori_loop(0, _NST // 2, hist_loop, 0)
    drain_stage(hrA, semA)

    pltpu.sync_copy(pool_v, x_out.at[bs, pl.ds(27 * _D, _D)])


def _tc_body(x_ref, y_ref, wu1, bu1, wu2, bu2, wi1, bi1, wi2, bi2,
             u_ref, i_ref):
    f32 = jnp.float32
    xb = x_ref[...]
    h = jnp.maximum(
        jnp.dot(xb, wu1[...], preferred_element_type=f32) + bu1[...], 0.0)
    uu = jnp.dot(h, wu2[...], preferred_element_type=f32) + bu2[...]
    n = jnp.sqrt(jnp.sum(uu * uu, axis=-1, keepdims=True))
    u_ref[...] = uu / jnp.maximum(n, 1e-12)

    yb = y_ref[...]
    h2 = jnp.maximum(
        jnp.dot(yb, wi1[...], preferred_element_type=f32) + bi1[...], 0.0)
    ii = jnp.dot(h2, wi2[...], preferred_element_type=f32) + bi2[...]
    n2 = jnp.sqrt(jnp.sum(ii * ii, axis=-1, keepdims=True))
    i_ref[...] = ii / jnp.maximum(n2, 1e-12)


def kernel(user_cat, user_num, ctx_cat, hist_ids, hist_mask, item_cat,
           item_num, user_tables, ctx_tables, item_table0, item_tables_rest,
           Wun, bun, Win, bin, Wu1, bu1, Wu2, bu2, Wi1, bi1, Wi2, bi2):
    f32 = jnp.float32
    ucat_f = user_cat.T.astype(jnp.int32).reshape(-1)
    ccat_f = ctx_cat.T.astype(jnp.int32).reshape(-1)
    icat_f = item_cat.T.astype(jnp.int32).reshape(-1)
    hist_flat = hist_ids.reshape(-1).astype(jnp.int32)
    hmask_flat = hist_mask.reshape(-1)
    unum_pad = jnp.pad(user_num, ((0, 0), (0, _D - _UNUM))).reshape(-1)
    inum_pad = jnp.pad(item_num, ((0, 0), (0, _D - _INUM))).reshape(-1)

    utT = jnp.swapaxes(user_tables, 1, 2)
    irT = jnp.swapaxes(item_tables_rest, 1, 2)

    mesh = plsc.VectorSubcoreMesh(core_axis_name="c", subcore_axis_name="s")
    repack = functools.partial(
        pl.kernel,
        mesh=mesh,
        out_type=[jax.ShapeDtypeStruct((_VI0 * _D,), f32),
                  jax.ShapeDtypeStruct((_NU * _VU * _D,), f32),
                  jax.ShapeDtypeStruct(((_NI - 1) * _VIR * _D,), f32)],
        scratch_types=[
            pltpu.VMEM((128, _D), f32),
            pltpu.VMEM((16, _CW), f32),
            pltpu.VMEM((16, _CW), f32),
            pltpu.VMEM((16, _TW), f32),
            pltpu.VMEM((2048,), f32),
            pltpu.VMEM((16 * _CW,), f32),
        ],
    )(_repack_body)
    it0L, utL, irL = repack(item_table0, utT, irT)
    ct_flat = ctx_tables.reshape(_NC * _VC, _D)

    it0_lin = it0L.reshape(_VI0, _D)

    sc = functools.partial(
        pl.kernel,
        mesh=mesh,
        compiler_params=pltpu.CompilerParams(use_tc_tiling_on_sc=False),
        out_type=[jax.ShapeDtypeStruct((_B, _UIN), f32),
                  jax.ShapeDtypeStruct((_B, _IIN), f32)],
        scratch_types=[
            pltpu.VMEM((_RPT,), jnp.int32),
            pltpu.VMEM((_RPT,), jnp.int32),
            pltpu.VMEM((16 * _RPT,), jnp.int32),
            pltpu.VMEM((16 * _RPT,), jnp.int32),
            pltpu.VMEM((16 * _RPT,), f32),
            pltpu.VMEM((16 * _RPT,), f32),
            pltpu.VMEM((_RPT, _D), f32),
            pltpu.VMEM((_RPT, _D), f32),
            pltpu.VMEM((_RPT * _HL,), jnp.int32),
            pltpu.VMEM((_RPT * _HL,), f32),
            pltpu.VMEM((_SID, _D), f32),
            pltpu.VMEM((_SID, _D), f32),
            pltpu.VMEM((_RPT, _D), f32),
            pltpu.VMEM((_RPT, _D), f32),
            pltpu.VMEM((_RPT * _D,), f32),
            pltpu.VMEM((_RPT * _D,), f32),
            pltpu.VMEM((_UNUM * _D,), f32),
            pltpu.VMEM((_D,), f32),
            pltpu.VMEM((_INUM * _D,), f32),
            pltpu.VMEM((_D,), f32),
            pltpu.SemaphoreType.DMA,
            pltpu.SemaphoreType.DMA,
        ],
    )(_sc_body)
    x, y = sc(ucat_f, ccat_f, icat_f, hist_flat, hmask_flat, unum_pad,
              inum_pad, Wun.reshape(-1), bun, Win.reshape(-1), bin,
              utL, ct_flat, it0_lin, irL)

    bm = 1024
    grid = _B // bm
    full = lambda i: (0, 0)
    u, i = pl.pallas_call(
        _tc_body,
        grid=(grid,),
        in_specs=[
            pl.BlockSpec((bm, _UIN), lambda i: (i, 0)),
            pl.BlockSpec((bm, _IIN), lambda i: (i, 0)),
            pl.BlockSpec((_UIN, _HID), full),
            pl.BlockSpec((1, _HID), full),
            pl.BlockSpec((_HID, _TOW), full),
            pl.BlockSpec((1, _TOW), full),
            pl.BlockSpec((_IIN, _HID), full),
            pl.BlockSpec((1, _HID), full),
            pl.BlockSpec((_HID, _TOW), full),
            pl.BlockSpec((1, _TOW), full),
        ],
        out_specs=[pl.BlockSpec((bm, _TOW), lambda i: (i, 0)),
                   pl.BlockSpec((bm, _TOW), lambda i: (i, 0))],
        out_shape=[jax.ShapeDtypeStruct((_B, _TOW), f32),
                   jax.ShapeDtypeStruct((_B, _TOW), f32)],
    )(x, y, Wu1, bu1.reshape(1, _HID), Wu2, bu2.reshape(1, _TOW),
      Wi1, bi1.reshape(1, _HID), Wi2, bi2.reshape(1, _TOW))
    return (u, i)


# speedup vs baseline: 1.2450x; 1.2450x over previous
import functools

import jax
import jax.numpy as jnp
from jax import lax
from jax.experimental import pallas as pl
from jax.experimental.pallas import tpu as pltpu
from jax.experimental.pallas import tpu_sc as plsc

_B = 4096
_D = 16
_NU = 23
_NC = 3
_NI = 8
_HL = 200
_VU = 100000
_VC = 1000
_VI0 = 1000000
_VIR = 100000
_UNUM = 4
_INUM = 6
_HID = 128
_TOW = 64
_UIN = _NU * _D + _NC * _D + 2 * _D
_IIN = _NI * _D + _D

_NW = 32
_RPT = _B // _NW
_RPS = 4
_NST = _RPT // _NW * _NW // _RPS * 1
_NST = _RPT // _RPS
_SID = _RPS * _HL

_CW = 1024
_NCH = _VU // _CW
_TW = _VU - _NCH * _CW
_NCP = _NCH + 1
_CB = 16 * _CW


def _repack_body(it0, utT, irT, it0L, utL, irL,
                 srcvA, srcvB, colvA, colvB, tailv, stgA, stgB,
                 stg2A, stg2B, semA, semB, semC, semD):
    info = plsc.get_sparse_core_info()
    wid = lax.axis_index("s") * info.num_cores + lax.axis_index("c")

    nfull = _VI0 // 128

    def a_issue_in(c, srcv, sem):
        @pl.when(c < nfull)
        def _():
            r0 = pl.multiple_of(c * 128, 8)
            pltpu.async_copy(it0.at[pl.ds(r0, 128), :], srcv, sem)

    def a_drain_in(srcv, sem):
        pltpu.make_async_copy(it0.at[pl.ds(0, 128), :], srcv, sem).wait()

    def a_proc(c, srcv, stg, sem, osem):
        @pl.when(c < nfull)
        def _():
            a_drain_in(srcv, sem)

            def rbody(r, cc):
                stg[pl.ds(pl.multiple_of(r * 16, 16), 16)] = srcv[r, :]
                return cc
            lax.fori_loop(0, 128, rbody, 0)
            pltpu.async_copy(stg, it0L.at[pl.ds(c * 2048, 2048)], osem)

    def a_drain_out(c, stg, osem):
        @pl.when(c < nfull)
        def _():
            pltpu.make_async_copy(it0L.at[pl.ds(0, 2048)], stg,
                                  osem).wait()

    a_issue_in(wid, srcvA, semA)

    def it0_loop(t, carry):
        cA = wid + 2 * t * _NW
        cB = wid + (2 * t + 1) * _NW
        a_issue_in(cB, srcvB, semB)
        a_proc(cA, srcvA, stgA, semA, semC)
        a_issue_in(cA + 2 * _NW, srcvA, semA)
        a_proc(cB, srcvB, stgB, semB, semD)
        a_drain_out(cA, stgA, semC)
        a_drain_out(cB, stgB, semD)
        return carry
    lax.fori_loop(0, nfull // (2 * _NW) + 1, it0_loop, 0)

    @pl.when(wid == 0)
    def _():
        r0 = nfull * 128
        pltpu.sync_copy(it0.at[pl.ds(r0, 64), :],
                        srcvA.at[pl.ds(0, 64), :])

        def rbody(r, cc):
            stgA[pl.ds(pl.multiple_of(r * 16, 16), 16)] = srcvA[r, :]
            return cc
        lax.fori_loop(0, 64, rbody, 0)
        pltpu.sync_copy(stgA.at[pl.ds(0, 1024)],
                        it0L.at[pl.ds(r0 * 16, 1024)])

    def b_issue_in(src3, f, i0, cw, buf, sem):
        pltpu.sync_copy(src3.at[f, pl.ds(0, 8), pl.ds(i0, cw)],
                        buf.at[pl.ds(0, 8), pl.ds(0, cw)])
        pltpu.sync_copy(src3.at[f, pl.ds(8, 8), pl.ds(i0, cw)],
                        buf.at[pl.ds(8, 8), pl.ds(0, cw)])

    def b_rows(buf, stg2, nc):
        def jrow(j, carry):
            def cbody(c, cc):
                o = pl.multiple_of(c * 16, 16)
                stg2[pl.ds(pl.multiple_of(j * _CW + o, 16), 16)] = \
                    buf[j, pl.ds(o, 16)]
                return cc
            lax.fori_loop(0, nc, cbody, 0)
            return carry
        lax.fori_loop(0, 16, jrow, 0)

    def b_out(out1d, f, c, stg2, sem):
        off = pl.multiple_of((f * _NCP + c) * _CB, 8)
        pltpu.async_copy(stg2, out1d.at[pl.ds(off, _CB)], sem)

    def b_drain_out(stg2, sem):
        pltpu.make_async_copy(utL.at[pl.ds(0, _CB)], stg2, sem).wait()

    def b_phase(src3, out1d, nplanes):
        ntask = nplanes * _NCH

        def b_proc(c, buf, stg2, sem, first):
            @pl.when(c < ntask)
            def _():
                f = c // _NCH
                i0 = pl.multiple_of((c - f * _NCH) * _CW, 128)
                b_issue_in(src3, f, i0, _CW, buf, sem)
                if not first:
                    b_drain_out(stg2, sem)
                b_rows(buf, stg2, _CW // 16)
                b_out(out1d, f, c - f * _NCH, stg2, sem)

        b_proc(wid, colvA, stg2A, semA, True)
        b_proc(wid + _NW, colvB, stg2B, semB, True)

        def loop(t, carry):
            cA = wid + (2 * t + 2) * _NW
            cB = wid + (2 * t + 3) * _NW
            b_proc(cA, colvA, stg2A, semA, False)
            b_proc(cB, colvB, stg2B, semB, False)
            return carry
        lax.fori_loop(0, ntask // (2 * _NW) + 1, loop, 0)

        @pl.when(wid < ntask)
        def _():
            b_drain_out(stg2A, semA)

        @pl.when(wid + _NW < ntask)
        def _():
            b_drain_out(stg2B, semB)

    b_phase(utT, utL, _NU)
    b_phase(irT, irL, _NI - 1)

    def b_tail(src3, out1d, f):
        b_issue_in(src3, f, _NCH * _CW, _TW, tailv, semA)
        b_rows(tailv, stg2A, _TW // 16)
        b_out(out1d, f, _NCH, stg2A, semA)
        b_drain_out(stg2A, semA)

    @pl.when(wid < _NU)
    def _():
        b_tail(utT, utL, wid)

    @pl.when(jnp.logical_and(wid >= _NU, wid < _NU + _NI - 1))
    def _():
        b_tail(irT, irL, wid - _NU)


def _sc_body(ucat, ccat, icat, histf, hmaskf, unum, inum, wun, bun2, win,
             bin2, utf, ctf, it0, irf, x_out, y_out,
             idxA, idxB, idxTA, idxTB, colgA, colgB, embA, embB,
             ids_all, hmask_all, hrA, hrB,
             pool_v, num_v, un_v, in_v, wun_v, bun_v, win_v, bin_v,
             semA, semB):
    info = plsc.get_sparse_core_info()
    wid = lax.axis_index("s") * info.num_cores + lax.axis_index("c")
    b0 = wid * _RPT
    bs = pl.ds(b0, _RPT)

    pltpu.sync_copy(wun, wun_v)
    pltpu.sync_copy(bun2, bun_v)
    pltpu.sync_copy(win, win_v)
    pltpu.sync_copy(bin2, bin_v)
    pltpu.sync_copy(unum.at[pl.ds(b0 * _D, _RPT * _D)], un_v)
    pltpu.sync_copy(inum.at[pl.ds(b0 * _D, _RPT * _D)], in_v)

    def unum_body(r, carry):
        uvec = un_v[pl.ds(pl.multiple_of(r * _D, _D), _D)]
        acc = bun_v[...]
        for k in range(_UNUM):
            acc = acc + uvec[k] * wun_v[pl.ds(k * _D, _D)]
        num_v[r, :] = acc
        return carry
    lax.fori_loop(0, _RPT, unum_body, 0)
    pltpu.sync_copy(num_v, x_out.at[bs, pl.ds(26 * _D, _D)])

    def inum_body(r, carry):
        ivec = in_v[pl.ds(pl.multiple_of(r * _D, _D), _D)]
        acc = bin_v[...]
        for k in range(_INUM):
            acc = acc + ivec[k] * win_v[pl.ds(k * _D, _D)]
        num_v[r, :] = acc
        return carry
    lax.fori_loop(0, _RPT, inum_body, 0)
    pltpu.sync_copy(num_v, y_out.at[bs, pl.ds(_NI * _D, _D)])

    i16 = lax.iota(jnp.int32, 16)

    def wprep(srcarr, srcf, f, table, idx_v, idxT_v, colg_v, sem):
        pltpu.sync_copy(
            srcarr.at[pl.ds(pl.multiple_of(srcf * _B + b0, 8), _RPT)],
            idx_v)
        jv = i16 * _CW

        def rxf(c, carry):
            idv = idx_v[pl.ds(pl.multiple_of(c * 16, 16), 16)]
            bvec = ((idv >> 10) * _CB + (idv & 1023)
                    + f * (_NCP * _CB))
            for m in range(16):
                r = c * 16 + m
                idxT_v[pl.ds(pl.multiple_of(r * 16, 16), 16)] = \
                    jv + bvec[m]
            return carry
        lax.fori_loop(0, _RPT // 16, rxf, 0)

        def jissue(j, carry):
            jb = pl.multiple_of(j * 128, 128)
            pltpu.async_copy(table.at[idxT_v.at[pl.ds(jb, 128)]],
                             colg_v.at[pl.ds(jb, 128)], sem)
            return carry
        lax.fori_loop(0, 16, jissue, 0)

    def wfinish(dstbuf, col, colg_v, emb_v, sem):
        def jdrain(j, carry):
            jb = pl.multiple_of(j * 128, 128)
            pltpu.make_async_copy(utf.at[pl.ds(0, 128)],
                                  colg_v.at[pl.ds(jb, 128)], sem).wait()
            return carry
        lax.fori_loop(0, 16, jdrain, 0)

        def rtrans(rb, carry):
            for m in range(16):
                r = rb * 16 + m
                emb_v[r, :] = colg_v[pl.ds(pl.multiple_of(r * 16, 16), 16)]
            return carry
        lax.fori_loop(0, _RPT // 16, rtrans, 0)
        pltpu.sync_copy(emb_v,
                        dstbuf.at[bs, pl.ds(pl.multiple_of(col, 16), _D)])

    def ugrp(t, carry):
        f1 = 2 * t
        f2 = 2 * t + 1
        wprep(ucat, f1, f1, utf, idxA, idxTA, colgA, semA)

        @pl.when(f2 < _NU)
        def _():
            wprep(ucat, f2, f2, utf, idxB, idxTB, colgB, semB)
        wfinish(x_out, f1 * _D, colgA, embA, semA)

        @pl.when(f2 < _NU)
        def _():
            wfinish(x_out, f2 * _D, colgB, embB, semB)
        return carry
    lax.fori_loop(0, (_NU + 1) // 2, ugrp, 0)

    def igrp2(t, carry):
        f1 = 2 * t
        f2 = 2 * t + 1
        wprep(icat, f1 + 1, f1, irf, idxA, idxTA, colgA, semA)

        @pl.when(f2 < _NI - 1)
        def _():
            wprep(icat, f2 + 1, f2, irf, idxB, idxTB, colgB, semB)
        wfinish(y_out, (f1 + 1) * _D, colgA, embA, semA)

        @pl.when(f2 < _NI - 1)
        def _():
            wfinish(y_out, (f2 + 1) * _D, colgB, embB, semB)
        return carry
    lax.fori_loop(0, _NI // 2, igrp2, 0)

    for f in range(_NC):
        pltpu.sync_copy(ccat.at[pl.ds(f * _B + b0, _RPT)], idxA)
        if f:
            for c in range(_RPT // 16):
                sl = pl.ds(c * 16, 16)
                idxA[sl] = idxA[sl] + f * _VC
        pltpu.async_copy(ctf.at[idxA], embA, semA).wait()
        pltpu.sync_copy(embA, x_out.at[bs, pl.ds((_NU + f) * _D, _D)])

    pltpu.sync_copy(icat.at[pl.ds(b0, _RPT)], idxA)
    pltpu.async_copy(it0.at[idxA], embA, semA).wait()
    pltpu.sync_copy(embA, y_out.at[bs, pl.ds(0, _D)])

    pltpu.sync_copy(histf.at[pl.ds(b0 * _HL, _RPT * _HL)], ids_all)
    pltpu.sync_copy(hmaskf.at[pl.ds(b0 * _HL, _RPT * _HL)], hmask_all)

    def issue_stage(s, buf, sem):
        for k in range(_RPS):
            o = pl.multiple_of(s * _SID + k * _HL, 8)
            pltpu.async_copy(it0.at[ids_all.at[pl.ds(o, 128)]],
                             buf.at[pl.ds(k * _HL, 128)], sem)
            pltpu.async_copy(it0.at[ids_all.at[pl.ds(o + 128, _HL - 128)]],
                             buf.at[pl.ds(k * _HL + 128, _HL - 128)], sem)

    def drain_stage(buf, sem):
        for k in range(_RPS):
            pltpu.make_async_copy(it0.at[pl.ds(0, 128)],
                                  buf.at[pl.ds(k * _HL, 128)], sem).wait()
            pltpu.make_async_copy(it0.at[pl.ds(0, _HL - 128)],
                                  buf.at[pl.ds(k * _HL + 128, _HL - 128)],
                                  sem).wait()

    def compute_stage(s, buf):
        for k in range(_RPS):
            mbase = s * _SID + k * _HL
            zv = jnp.zeros((16,), jnp.float32)

            def acc_body(c, carry2):
                accs, ms = carry2
                accs = list(accs)
                mvec = hmask_all[pl.ds(pl.multiple_of(mbase + c * 16, 8), 16)]
                base = k * _HL + c * 16
                for j in range(16):
                    mj = mvec[j]
                    accs[j % 4] = accs[j % 4] + buf[base + j, :] * mj
                    ms = ms + mj
                return (tuple(accs), ms)
            accs, ms = lax.fori_loop(
                0, 12, acc_body, ((zv, zv, zv, zv), jnp.float32(0.0)))
            a0, a1, a2, a3 = accs
            mvec = hmask_all[pl.ds(pl.multiple_of(mbase + 192, 8), 16)]
            for j in range(8):
                mj = mvec[j]
                a0 = a0 + buf[k * _HL + 192 + j, :] * mj
                ms = ms + mj
            a = (a0 + a1) + (a2 + a3)
            pool_v[s * _RPS + k, :] = a / jnp.maximum(ms, 1e-6)

    issue_stage(0, hrA, semA)

    def hist_loop(t, carry):
        sA = 2 * t
        sB = 2 * t + 1
        issue_stage(sB, hrB, semB)
        drain_stage(hrA, semA)
        compute_stage(sA, hrA)
        issue_stage(lax.rem(sA + 2, _NST), hrA, semA)
        drain_stage(hrB, semB)
        compute_stage(sB, hrB)
        return carry
    lax.fori_loop(0, _NST // 2, hist_loop, 0)
    drain_stage(hrA, semA)

    pltpu.sync_copy(pool_v, x_out.at[bs, pl.ds(27 * _D, _D)])


def _tc_body(x_ref, y_ref, wu1, bu1, wu2, bu2, wi1, bi1, wi2, bi2,
             u_ref, i_ref):
    f32 = jnp.float32
    xb = x_ref[...]
    h = jnp.maximum(
        jnp.dot(xb, wu1[...], preferred_element_type=f32) + bu1[...], 0.0)
    uu = jnp.dot(h, wu2[...], preferred_element_type=f32) + bu2[...]
    n = jnp.sqrt(jnp.sum(uu * uu, axis=-1, keepdims=True))
    u_ref[...] = uu / jnp.maximum(n, 1e-12)

    yb = y_ref[...]
    h2 = jnp.maximum(
        jnp.dot(yb, wi1[...], preferred_element_type=f32) + bi1[...], 0.0)
    ii = jnp.dot(h2, wi2[...], preferred_element_type=f32) + bi2[...]
    n2 = jnp.sqrt(jnp.sum(ii * ii, axis=-1, keepdims=True))
    i_ref[...] = ii / jnp.maximum(n2, 1e-12)


def kernel(user_cat, user_num, ctx_cat, hist_ids, hist_mask, item_cat,
           item_num, user_tables, ctx_tables, item_table0, item_tables_rest,
           Wun, bun, Win, bin, Wu1, bu1, Wu2, bu2, Wi1, bi1, Wi2, bi2):
    f32 = jnp.float32
    ucat_f = user_cat.T.astype(jnp.int32).reshape(-1)
    ccat_f = ctx_cat.T.astype(jnp.int32).reshape(-1)
    icat_f = item_cat.T.astype(jnp.int32).reshape(-1)
    hist_flat = hist_ids.reshape(-1).astype(jnp.int32)
    hmask_flat = hist_mask.reshape(-1)
    unum_pad = jnp.pad(user_num, ((0, 0), (0, _D - _UNUM))).reshape(-1)
    inum_pad = jnp.pad(item_num, ((0, 0), (0, _D - _INUM))).reshape(-1)

    utT = jnp.swapaxes(user_tables, 1, 2)
    irT = jnp.swapaxes(item_tables_rest, 1, 2)

    mesh = plsc.VectorSubcoreMesh(core_axis_name="c", subcore_axis_name="s")
    repack = functools.partial(
        pl.kernel,
        mesh=mesh,
        out_type=[jax.ShapeDtypeStruct((_VI0 * _D,), f32),
                  jax.ShapeDtypeStruct((_NU * _NCP * _CB,), f32),
                  jax.ShapeDtypeStruct(((_NI - 1) * _NCP * _CB,), f32)],
        scratch_types=[
            pltpu.VMEM((128, _D), f32),
            pltpu.VMEM((128, _D), f32),
            pltpu.VMEM((16, _CW), f32),
            pltpu.VMEM((16, _CW), f32),
            pltpu.VMEM((16, _TW), f32),
            pltpu.VMEM((2048,), f32),
            pltpu.VMEM((2048,), f32),
            pltpu.VMEM((_CB,), f32),
            pltpu.VMEM((_CB,), f32),
            pltpu.SemaphoreType.DMA,
            pltpu.SemaphoreType.DMA,
            pltpu.SemaphoreType.DMA,
            pltpu.SemaphoreType.DMA,
        ],
    )(_repack_body)
    it0L, utL, irL = repack(item_table0, utT, irT)
    ct_flat = ctx_tables.reshape(_NC * _VC, _D)

    it0_lin = it0L.reshape(_VI0, _D)

    sc = functools.partial(
        pl.kernel,
        mesh=mesh,
        compiler_params=pltpu.CompilerParams(use_tc_tiling_on_sc=False),
        out_type=[jax.ShapeDtypeStruct((_B, _UIN), f32),
                  jax.ShapeDtypeStruct((_B, _IIN), f32)],
        scratch_types=[
            pltpu.VMEM((_RPT,), jnp.int32),
            pltpu.VMEM((_RPT,), jnp.int32),
            pltpu.VMEM((16 * _RPT,), jnp.int32),
            pltpu.VMEM((16 * _RPT,), jnp.int32),
            pltpu.VMEM((16 * _RPT,), f32),
            pltpu.VMEM((16 * _RPT,), f32),
            pltpu.VMEM((_RPT, _D), f32),
            pltpu.VMEM((_RPT, _D), f32),
            pltpu.VMEM((_RPT * _HL,), jnp.int32),
            pltpu.VMEM((_RPT * _HL,), f32),
            pltpu.VMEM((_SID, _D), f32),
            pltpu.VMEM((_SID, _D), f32),
            pltpu.VMEM((_RPT, _D), f32),
            pltpu.VMEM((_RPT, _D), f32),
            pltpu.VMEM((_RPT * _D,), f32),
            pltpu.VMEM((_RPT * _D,), f32),
            pltpu.VMEM((_UNUM * _D,), f32),
            pltpu.VMEM((_D,), f32),
            pltpu.VMEM((_INUM * _D,), f32),
            pltpu.VMEM((_D,), f32),
            pltpu.SemaphoreType.DMA,
            pltpu.SemaphoreType.DMA,
        ],
    )(_sc_body)
    x, y = sc(ucat_f, ccat_f, icat_f, hist_flat, hmask_flat, unum_pad,
              inum_pad, Wun.reshape(-1), bun, Win.reshape(-1), bin,
              utL, ct_flat, it0_lin, irL)

    bm = 1024
    grid = _B // bm
    full = lambda i: (0, 0)
    u, i = pl.pallas_call(
        _tc_body,
        grid=(grid,),
        in_specs=[
            pl.BlockSpec((bm, _UIN), lambda i: (i, 0)),
            pl.BlockSpec((bm, _IIN), lambda i: (i, 0)),
            pl.BlockSpec((_UIN, _HID), full),
            pl.BlockSpec((1, _HID), full),
            pl.BlockSpec((_HID, _TOW), full),
            pl.BlockSpec((1, _TOW), full),
            pl.BlockSpec((_IIN, _HID), full),
            pl.BlockSpec((1, _HID), full),
            pl.BlockSpec((_HID, _TOW), full),
            pl.BlockSpec((1, _TOW), full),
        ],
        out_specs=[pl.BlockSpec((bm, _TOW), lambda i: (i, 0)),
                   pl.BlockSpec((bm, _TOW), lambda i: (i, 0))],
        out_shape=[jax.ShapeDtypeStruct((_B, _TOW), f32),
                   jax.ShapeDtypeStruct((_B, _TOW), f32)],
    )(x, y, Wu1, bu1.reshape(1, _HID), Wu2, bu2.reshape(1, _TOW),
      Wi1, bi1.reshape(1, _HID), Wi2, bi2.reshape(1, _TOW))
    return (u, i)


# speedup vs baseline: 1.4912x; 1.1978x over previous
import functools

import jax
import jax.numpy as jnp
from jax import lax
from jax.experimental import pallas as pl
from jax.experimental.pallas import tpu as pltpu
from jax.experimental.pallas import tpu_sc as plsc

_B = 4096
_D = 16
_NU = 23
_NC = 3
_NI = 8
_HL = 200
_VU = 100000
_VC = 1000
_VI0 = 1000000
_VIR = 100000
_UNUM = 4
_INUM = 6
_HID = 128
_TOW = 64
_UIN = _NU * _D + _NC * _D + 2 * _D
_IIN = _NI * _D + _D

_NW = 32
_RPT = _B // _NW
_RPS = 4
_NST = _RPT // _NW * _NW // _RPS * 1
_NST = _RPT // _RPS
_SID = _RPS * _HL

_CW = 1024
_NCH = _VU // _CW
_TW = _VU - _NCH * _CW
_NCP = _NCH + 1
_CB = 16 * _CW


def _repack_body(it0, utT, irT, it0L, utL, irL,
                 srcvA, srcvB, colvA, colvB, tailv, stgA, stgB,
                 stg2A, stg2B, semA, semB, semC, semD):
    info = plsc.get_sparse_core_info()
    wid = lax.axis_index("s") * info.num_cores + lax.axis_index("c")

    nfull = _VI0 // 128

    def a_issue_in(c, srcv, sem):
        @pl.when(c < nfull)
        def _():
            r0 = pl.multiple_of(c * 128, 8)
            pltpu.async_copy(it0.at[pl.ds(r0, 128), :], srcv, sem)

    def a_drain_in(srcv, sem):
        pltpu.make_async_copy(it0.at[pl.ds(0, 128), :], srcv, sem).wait()

    def a_proc(c, srcv, stg, sem, osem):
        @pl.when(c < nfull)
        def _():
            a_drain_in(srcv, sem)

            def rbody(r, cc):
                stg[pl.ds(pl.multiple_of(r * 16, 16), 16)] = srcv[r, :]
                return cc
            lax.fori_loop(0, 128, rbody, 0)
            pltpu.async_copy(stg, it0L.at[pl.ds(c * 2048, 2048)], osem)

    def a_drain_out(c, stg, osem):
        @pl.when(c < nfull)
        def _():
            pltpu.make_async_copy(it0L.at[pl.ds(0, 2048)], stg,
                                  osem).wait()

    a_issue_in(wid, srcvA, semA)

    def it0_loop(t, carry):
        cA = wid + 2 * t * _NW
        cB = wid + (2 * t + 1) * _NW
        a_issue_in(cB, srcvB, semB)
        a_proc(cA, srcvA, stgA, semA, semC)
        a_issue_in(cA + 2 * _NW, srcvA, semA)
        a_proc(cB, srcvB, stgB, semB, semD)
        a_drain_out(cA, stgA, semC)
        a_drain_out(cB, stgB, semD)
        return carry
    lax.fori_loop(0, nfull // (2 * _NW) + 1, it0_loop, 0)

    @pl.when(wid == 0)
    def _():
        r0 = nfull * 128
        pltpu.sync_copy(it0.at[pl.ds(r0, 64), :],
                        srcvA.at[pl.ds(0, 64), :])

        def rbody(r, cc):
            stgA[pl.ds(pl.multiple_of(r * 16, 16), 16)] = srcvA[r, :]
            return cc
        lax.fori_loop(0, 64, rbody, 0)
        pltpu.sync_copy(stgA.at[pl.ds(0, 1024)],
                        it0L.at[pl.ds(r0 * 16, 1024)])

    def b_issue_in(src3, f, i0, cw, buf, sem):
        pltpu.async_copy(src3.at[f, pl.ds(0, 8), pl.ds(i0, cw)],
                         buf.at[pl.ds(0, 8), pl.ds(0, cw)], sem)
        pltpu.async_copy(src3.at[f, pl.ds(8, 8), pl.ds(i0, cw)],
                         buf.at[pl.ds(8, 8), pl.ds(0, cw)], sem)

    def b_drain_in(src3, buf, i0, cw, sem):
        pltpu.make_async_copy(src3.at[0, pl.ds(0, 8), pl.ds(i0, cw)],
                              buf.at[pl.ds(0, 8), pl.ds(0, cw)],
                              sem).wait()
        pltpu.make_async_copy(src3.at[0, pl.ds(8, 8), pl.ds(i0, cw)],
                              buf.at[pl.ds(8, 8), pl.ds(0, cw)],
                              sem).wait()

    def b_rows(buf, stg2, nc):
        def jrow(j, carry):
            def cbody(c, cc):
                o = pl.multiple_of(c * 16, 16)
                stg2[pl.ds(pl.multiple_of(j * _CW + o, 16), 16)] = \
                    buf[j, pl.ds(o, 16)]
                return cc
            lax.fori_loop(0, nc, cbody, 0)
            return carry
        lax.fori_loop(0, 16, jrow, 0)

    def b_out(out1d, f, c, stg2, sem):
        off = pl.multiple_of((f * _NCP + c) * _CB, 8)
        pltpu.async_copy(stg2, out1d.at[pl.ds(off, _CB)], sem)

    def b_drain_out(stg2, sem):
        pltpu.make_async_copy(utL.at[pl.ds(0, _CB)], stg2, sem).wait()

    def b_phase(src3, out1d, nplanes):
        ntask = nplanes * _NCH

        def task_in(c, buf, insem):
            @pl.when(c < ntask)
            def _():
                f = c // _NCH
                i0 = pl.multiple_of((c - f * _NCH) * _CW, 128)
                b_issue_in(src3, f, i0, _CW, buf, insem)

        def b_proc(c, t, buf, stg2, insem, osem):
            @pl.when(c < ntask)
            def _():
                b_drain_in(src3, buf, 0, _CW, insem)

                @pl.when(t > 0)
                def _():
                    b_drain_out(stg2, osem)
                b_rows(buf, stg2, _CW // 16)
                f = c // _NCH
                b_out(out1d, f, c - f * _NCH, stg2, osem)

        task_in(wid, colvA, semA)

        def loop(t, carry):
            cA = wid + 2 * t * _NW
            cB = wid + (2 * t + 1) * _NW
            task_in(cB, colvB, semB)
            b_proc(cA, t, colvA, stg2A, semA, semC)
            task_in(cA + 2 * _NW, colvA, semA)
            b_proc(cB, t, colvB, stg2B, semB, semD)
            return carry
        lax.fori_loop(0, ntask // (2 * _NW) + 1, loop, 0)

        @pl.when(wid < ntask)
        def _():
            b_drain_out(stg2A, semC)

        @pl.when(wid + _NW < ntask)
        def _():
            b_drain_out(stg2B, semD)

    b_phase(utT, utL, _NU)
    b_phase(irT, irL, _NI - 1)

    def b_tail(src3, out1d, f):
        b_issue_in(src3, f, _NCH * _CW, _TW, tailv, semA)
        b_drain_in(src3, tailv, _NCH * _CW, _TW, semA)
        b_rows(tailv, stg2A, _TW // 16)
        b_out(out1d, f, _NCH, stg2A, semA)
        b_drain_out(stg2A, semA)

    @pl.when(wid < _NU)
    def _():
        b_tail(utT, utL, wid)

    @pl.when(jnp.logical_and(wid >= _NU, wid < _NU + _NI - 1))
    def _():
        b_tail(irT, irL, wid - _NU)


def _sc_body(ucat, ccat, icat, histf, hmaskf, unum, inum, wun, bun2, win,
             bin2, utf, ctf, it0, irf, x_out, y_out,
             idxA, idxB, idxTA, idxTB, colgA, colgB, embA, embB,
             ids_all, hmask_all, hrA, hrB,
             pool_v, num_v, un_v, in_v, wun_v, bun_v, win_v, bin_v,
             semA, semB):
    info = plsc.get_sparse_core_info()
    wid = lax.axis_index("s") * info.num_cores + lax.axis_index("c")
    b0 = wid * _RPT
    bs = pl.ds(b0, _RPT)

    pltpu.sync_copy(wun, wun_v)
    pltpu.sync_copy(bun2, bun_v)
    pltpu.sync_copy(win, win_v)
    pltpu.sync_copy(bin2, bin_v)
    pltpu.sync_copy(unum.at[pl.ds(b0 * _D, _RPT * _D)], un_v)
    pltpu.sync_copy(inum.at[pl.ds(b0 * _D, _RPT * _D)], in_v)

    def unum_body(r, carry):
        uvec = un_v[pl.ds(pl.multiple_of(r * _D, _D), _D)]
        acc = bun_v[...]
        for k in range(_UNUM):
            acc = acc + uvec[k] * wun_v[pl.ds(k * _D, _D)]
        num_v[r, :] = acc
        return carry
    lax.fori_loop(0, _RPT, unum_body, 0)
    pltpu.sync_copy(num_v, x_out.at[bs, pl.ds(26 * _D, _D)])

    def inum_body(r, carry):
        ivec = in_v[pl.ds(pl.multiple_of(r * _D, _D), _D)]
        acc = bin_v[...]
        for k in range(_INUM):
            acc = acc + ivec[k] * win_v[pl.ds(k * _D, _D)]
        num_v[r, :] = acc
        return carry
    lax.fori_loop(0, _RPT, inum_body, 0)
    pltpu.sync_copy(num_v, y_out.at[bs, pl.ds(_NI * _D, _D)])

    i16 = lax.iota(jnp.int32, 16)

    def wprep(srcarr, srcf, f, table, idx_v, idxT_v, colg_v, sem):
        pltpu.sync_copy(
            srcarr.at[pl.ds(pl.multiple_of(srcf * _B + b0, 8), _RPT)],
            idx_v)
        jv = i16 * _CW

        def rxf(c, carry):
            idv = idx_v[pl.ds(pl.multiple_of(c * 16, 16), 16)]
            bvec = ((idv >> 10) * _CB + (idv & 1023)
                    + f * (_NCP * _CB))
            for m in range(16):
                r = c * 16 + m
                idxT_v[pl.ds(pl.multiple_of(r * 16, 16), 16)] = \
                    jv + bvec[m]
            return carry
        lax.fori_loop(0, _RPT // 16, rxf, 0)

        def jissue(j, carry):
            jb = pl.multiple_of(j * 128, 128)
            pltpu.async_copy(table.at[idxT_v.at[pl.ds(jb, 128)]],
                             colg_v.at[pl.ds(jb, 128)], sem)
            return carry
        lax.fori_loop(0, 16, jissue, 0)

    def wfinish(dstbuf, col, colg_v, emb_v, sem):
        def jdrain(j, carry):
            jb = pl.multiple_of(j * 128, 128)
            pltpu.make_async_copy(utf.at[pl.ds(0, 128)],
                                  colg_v.at[pl.ds(jb, 128)], sem).wait()
            return carry
        lax.fori_loop(0, 16, jdrain, 0)

        def rtrans(rb, carry):
            for m in range(16):
                r = rb * 16 + m
                emb_v[r, :] = colg_v[pl.ds(pl.multiple_of(r * 16, 16), 16)]
            return carry
        lax.fori_loop(0, _RPT // 16, rtrans, 0)
        pltpu.sync_copy(emb_v,
                        dstbuf.at[bs, pl.ds(pl.multiple_of(col, 16), _D)])

    def ugrp(t, carry):
        f1 = 2 * t
        f2 = 2 * t + 1
        wprep(ucat, f1, f1, utf, idxA, idxTA, colgA, semA)

        @pl.when(f2 < _NU)
        def _():
            wprep(ucat, f2, f2, utf, idxB, idxTB, colgB, semB)
        wfinish(x_out, f1 * _D, colgA, embA, semA)

        @pl.when(f2 < _NU)
        def _():
            wfinish(x_out, f2 * _D, colgB, embB, semB)
        return carry
    lax.fori_loop(0, (_NU + 1) // 2, ugrp, 0)

    def igrp2(t, carry):
        f1 = 2 * t
        f2 = 2 * t + 1
        wprep(icat, f1 + 1, f1, irf, idxA, idxTA, colgA, semA)

        @pl.when(f2 < _NI - 1)
        def _():
            wprep(icat, f2 + 1, f2, irf, idxB, idxTB, colgB, semB)
        wfinish(y_out, (f1 + 1) * _D, colgA, embA, semA)

        @pl.when(f2 < _NI - 1)
        def _():
            wfinish(y_out, (f2 + 1) * _D, colgB, embB, semB)
        return carry
    lax.fori_loop(0, _NI // 2, igrp2, 0)

    for f in range(_NC):
        pltpu.sync_copy(ccat.at[pl.ds(f * _B + b0, _RPT)], idxA)
        if f:
            for c in range(_RPT // 16):
                sl = pl.ds(c * 16, 16)
                idxA[sl] = idxA[sl] + f * _VC
        pltpu.async_copy(ctf.at[idxA], embA, semA).wait()
        pltpu.sync_copy(embA, x_out.at[bs, pl.ds((_NU + f) * _D, _D)])

    pltpu.sync_copy(icat.at[pl.ds(b0, _RPT)], idxA)
    pltpu.async_copy(it0.at[idxA], embA, semA).wait()
    pltpu.sync_copy(embA, y_out.at[bs, pl.ds(0, _D)])

    pltpu.sync_copy(histf.at[pl.ds(b0 * _HL, _RPT * _HL)], ids_all)
    pltpu.sync_copy(hmaskf.at[pl.ds(b0 * _HL, _RPT * _HL)], hmask_all)

    def issue_stage(s, buf, sem):
        for k in range(_RPS):
            o = pl.multiple_of(s * _SID + k * _HL, 8)
            pltpu.async_copy(it0.at[ids_all.at[pl.ds(o, 128)]],
                             buf.at[pl.ds(k * _HL, 128)], sem)
            pltpu.async_copy(it0.at[ids_all.at[pl.ds(o + 128, _HL - 128)]],
                             buf.at[pl.ds(k * _HL + 128, _HL - 128)], sem)

    def drain_stage(buf, sem):
        for k in range(_RPS):
            pltpu.make_async_copy(it0.at[pl.ds(0, 128)],
                                  buf.at[pl.ds(k * _HL, 128)], sem).wait()
            pltpu.make_async_copy(it0.at[pl.ds(0, _HL - 128)],
                                  buf.at[pl.ds(k * _HL + 128, _HL - 128)],
                                  sem).wait()

    def compute_stage(s, buf):
        for k in range(_RPS):
            mbase = s * _SID + k * _HL
            zv = jnp.zeros((16,), jnp.float32)

            def acc_body(c, carry2):
                accs, ms = carry2
                accs = list(accs)
                mvec = hmask_all[pl.ds(pl.multiple_of(mbase + c * 16, 8), 16)]
                base = k * _HL + c * 16
                for j in range(16):
                    mj = mvec[j]
                    accs[j % 4] = accs[j % 4] + buf[base + j, :] * mj
                    ms = ms + mj
                return (tuple(accs), ms)
            accs, ms = lax.fori_loop(
                0, 12, acc_body, ((zv, zv, zv, zv), jnp.float32(0.0)))
            a0, a1, a2, a3 = accs
            mvec = hmask_all[pl.ds(pl.multiple_of(mbase + 192, 8), 16)]
            for j in range(8):
                mj = mvec[j]
                a0 = a0 + buf[k * _HL + 192 + j, :] * mj
                ms = ms + mj
            a = (a0 + a1) + (a2 + a3)
            pool_v[s * _RPS + k, :] = a / jnp.maximum(ms, 1e-6)

    issue_stage(0, hrA, semA)

    def hist_loop(t, carry):
        sA = 2 * t
        sB = 2 * t + 1
        issue_stage(sB, hrB, semB)
        drain_stage(hrA, semA)
        compute_stage(sA, hrA)
        issue_stage(lax.rem(sA + 2, _NST), hrA, semA)
        drain_stage(hrB, semB)
        compute_stage(sB, hrB)
        return carry
    lax.fori_loop(0, _NST // 2, hist_loop, 0)
    drain_stage(hrA, semA)

    pltpu.sync_copy(pool_v, x_out.at[bs, pl.ds(27 * _D, _D)])


def _tc_body(x_ref, y_ref, wu1, bu1, wu2, bu2, wi1, bi1, wi2, bi2,
             u_ref, i_ref):
    f32 = jnp.float32
    xb = x_ref[...]
    h = jnp.maximum(
        jnp.dot(xb, wu1[...], preferred_element_type=f32) + bu1[...], 0.0)
    uu = jnp.dot(h, wu2[...], preferred_element_type=f32) + bu2[...]
    n = jnp.sqrt(jnp.sum(uu * uu, axis=-1, keepdims=True))
    u_ref[...] = uu / jnp.maximum(n, 1e-12)

    yb = y_ref[...]
    h2 = jnp.maximum(
        jnp.dot(yb, wi1[...], preferred_element_type=f32) + bi1[...], 0.0)
    ii = jnp.dot(h2, wi2[...], preferred_element_type=f32) + bi2[...]
    n2 = jnp.sqrt(jnp.sum(ii * ii, axis=-1, keepdims=True))
    i_ref[...] = ii / jnp.maximum(n2, 1e-12)


def kernel(user_cat, user_num, ctx_cat, hist_ids, hist_mask, item_cat,
           item_num, user_tables, ctx_tables, item_table0, item_tables_rest,
           Wun, bun, Win, bin, Wu1, bu1, Wu2, bu2, Wi1, bi1, Wi2, bi2):
    f32 = jnp.float32
    ucat_f = user_cat.T.astype(jnp.int32).reshape(-1)
    ccat_f = ctx_cat.T.astype(jnp.int32).reshape(-1)
    icat_f = item_cat.T.astype(jnp.int32).reshape(-1)
    hist_flat = hist_ids.reshape(-1).astype(jnp.int32)
    hmask_flat = hist_mask.reshape(-1)
    unum_pad = jnp.pad(user_num, ((0, 0), (0, _D - _UNUM))).reshape(-1)
    inum_pad = jnp.pad(item_num, ((0, 0), (0, _D - _INUM))).reshape(-1)

    utT = jnp.swapaxes(user_tables, 1, 2)
    irT = jnp.swapaxes(item_tables_rest, 1, 2)

    mesh = plsc.VectorSubcoreMesh(core_axis_name="c", subcore_axis_name="s")
    repack = functools.partial(
        pl.kernel,
        mesh=mesh,
        out_type=[jax.ShapeDtypeStruct((_VI0 * _D,), f32),
                  jax.ShapeDtypeStruct((_NU * _NCP * _CB,), f32),
                  jax.ShapeDtypeStruct(((_NI - 1) * _NCP * _CB,), f32)],
        scratch_types=[
            pltpu.VMEM((128, _D), f32),
            pltpu.VMEM((128, _D), f32),
            pltpu.VMEM((16, _CW), f32),
            pltpu.VMEM((16, _CW), f32),
            pltpu.VMEM((16, _TW), f32),
            pltpu.VMEM((2048,), f32),
            pltpu.VMEM((2048,), f32),
            pltpu.VMEM((_CB,), f32),
            pltpu.VMEM((_CB,), f32),
            pltpu.SemaphoreType.DMA,
            pltpu.SemaphoreType.DMA,
            pltpu.SemaphoreType.DMA,
            pltpu.SemaphoreType.DMA,
        ],
    )(_repack_body)
    it0L, utL, irL = repack(item_table0, utT, irT)
    ct_flat = ctx_tables.reshape(_NC * _VC, _D)

    it0_lin = it0L.reshape(_VI0, _D)

    sc = functools.partial(
        pl.kernel,
        mesh=mesh,
        compiler_params=pltpu.CompilerParams(use_tc_tiling_on_sc=False),
        out_type=[jax.ShapeDtypeStruct((_B, _UIN), f32),
                  jax.ShapeDtypeStruct((_B, _IIN), f32)],
        scratch_types=[
            pltpu.VMEM((_RPT,), jnp.int32),
            pltpu.VMEM((_RPT,), jnp.int32),
            pltpu.VMEM((16 * _RPT,), jnp.int32),
            pltpu.VMEM((16 * _RPT,), jnp.int32),
            pltpu.VMEM((16 * _RPT,), f32),
            pltpu.VMEM((16 * _RPT,), f32),
            pltpu.VMEM((_RPT, _D), f32),
            pltpu.VMEM((_RPT, _D), f32),
            pltpu.VMEM((_RPT * _HL,), jnp.int32),
            pltpu.VMEM((_RPT * _HL,), f32),
            pltpu.VMEM((_SID, _D), f32),
            pltpu.VMEM((_SID, _D), f32),
            pltpu.VMEM((_RPT, _D), f32),
            pltpu.VMEM((_RPT, _D), f32),
            pltpu.VMEM((_RPT * _D,), f32),
            pltpu.VMEM((_RPT * _D,), f32),
            pltpu.VMEM((_UNUM * _D,), f32),
            pltpu.VMEM((_D,), f32),
            pltpu.VMEM((_INUM * _D,), f32),
            pltpu.VMEM((_D,), f32),
            pltpu.SemaphoreType.DMA,
            pltpu.SemaphoreType.DMA,
        ],
    )(_sc_body)
    x, y = sc(ucat_f, ccat_f, icat_f, hist_flat, hmask_flat, unum_pad,
              inum_pad, Wun.reshape(-1), bun, Win.reshape(-1), bin,
              utL, ct_flat, it0_lin, irL)

    bm = 1024
    grid = _B // bm
    full = lambda i: (0, 0)
    u, i = pl.pallas_call(
        _tc_body,
        grid=(grid,),
        in_specs=[
            pl.BlockSpec((bm, _UIN), lambda i: (i, 0)),
            pl.BlockSpec((bm, _IIN), lambda i: (i, 0)),
            pl.BlockSpec((_UIN, _HID), full),
            pl.BlockSpec((1, _HID), full),
            pl.BlockSpec((_HID, _TOW), full),
            pl.BlockSpec((1, _TOW), full),
            pl.BlockSpec((_IIN, _HID), full),
            pl.BlockSpec((1, _HID), full),
            pl.BlockSpec((_HID, _TOW), full),
            pl.BlockSpec((1, _TOW), full),
        ],
        out_specs=[pl.BlockSpec((bm, _TOW), lambda i: (i, 0)),
                   pl.BlockSpec((bm, _TOW), lambda i: (i, 0))],
        out_shape=[jax.ShapeDtypeStruct((_B, _TOW), f32),
                   jax.ShapeDtypeStruct((_B, _TOW), f32)],
    )(x, y, Wu1, bu1.reshape(1, _HID), Wu2, bu2.reshape(1, _TOW),
      Wi1, bi1.reshape(1, _HID), Wi2, bi2.reshape(1, _TOW))
    return (u, i)


# speedup vs baseline: 1.5669x; 1.0508x over previous
import functools

import jax
import jax.numpy as jnp
from jax import lax
from jax.experimental import pallas as pl
from jax.experimental.pallas import tpu as pltpu
from jax.experimental.pallas import tpu_sc as plsc

_B = 4096
_D = 16
_NU = 23
_NC = 3
_NI = 8
_HL = 200
_VU = 100000
_VC = 1000
_VI0 = 1000000
_VIR = 100000
_UNUM = 4
_INUM = 6
_HID = 128
_TOW = 64
_UIN = _NU * _D + _NC * _D + 2 * _D
_IIN = _NI * _D + _D

_NW = 32
_RPT = _B // _NW
_RPS = 4
_NST = _RPT // _NW * _NW // _RPS * 1
_NST = _RPT // _RPS
_SID = _RPS * _HL

_CW = 512
_NCH = _VU // _CW
_TW = _VU - _NCH * _CW
_NCP = _NCH + 1
_CB = 16 * _CW


def _repack_body(it0, utT, irT, it0L, utL, irL,
                 srcv0, srcv1, srcv2, srcv3, colvA, colvB, tailv,
                 stg0, stg1, stg2, stg3, stg2A, stg2B,
                 semA, semB, semC, semD, osem0, osem1, osem2, osem3):
    info = plsc.get_sparse_core_info()
    wid = lax.axis_index("s") * info.num_cores + lax.axis_index("c")
    srcvs = [srcv0, srcv1, srcv2, srcv3]
    stgs = [stg0, stg1, stg2, stg3]
    isems = [semA, semB, semC, semD]
    osems = [osem0, osem1, osem2, osem3]

    nfull = _VI0 // 128

    def a_issue_in(c, srcv, sem):
        @pl.when(c < nfull)
        def _():
            r0 = pl.multiple_of(c * 128, 8)
            pltpu.async_copy(it0.at[pl.ds(r0, 128), :], srcv, sem)

    def a_drain_in(srcv, sem):
        pltpu.make_async_copy(it0.at[pl.ds(0, 128), :], srcv, sem).wait()

    def a_proc(c, t, srcv, stg, sem, osem):
        @pl.when(c < nfull)
        def _():
            a_drain_in(srcv, sem)

            @pl.when(t > 0)
            def _():
                pltpu.make_async_copy(it0L.at[pl.ds(0, 2048)], stg,
                                      osem).wait()

            def rbody(r, cc):
                stg[pl.ds(pl.multiple_of(r * 16, 16), 16)] = srcv[r, :]
                return cc
            lax.fori_loop(0, 128, rbody, 0)
            pltpu.async_copy(stg, it0L.at[pl.ds(c * 2048, 2048)], osem)

    for k in range(4):
        a_issue_in(wid + k * _NW, srcvs[k], isems[k])

    def it0_loop(t, carry):
        for k in range(4):
            c = wid + (4 * t + k) * _NW
            a_proc(c, t, srcvs[k], stgs[k], isems[k], osems[k])
            a_issue_in(c + 4 * _NW, srcvs[k], isems[k])
        return carry
    lax.fori_loop(0, nfull // (4 * _NW) + 1, it0_loop, 0)
    for k in range(4):
        @pl.when(wid + k * _NW < nfull)
        def _(k=k):
            pltpu.make_async_copy(it0L.at[pl.ds(0, 2048)], stgs[k],
                                  osems[k]).wait()

    @pl.when(wid == 0)
    def _():
        r0 = nfull * 128
        pltpu.sync_copy(it0.at[pl.ds(r0, 64), :],
                        srcv0.at[pl.ds(0, 64), :])

        def rbody(r, cc):
            stg0[pl.ds(pl.multiple_of(r * 16, 16), 16)] = srcv0[r, :]
            return cc
        lax.fori_loop(0, 64, rbody, 0)
        pltpu.sync_copy(stg0.at[pl.ds(0, 1024)],
                        it0L.at[pl.ds(r0 * 16, 1024)])

    def b_issue_in(src3, f, i0, cw, buf, sem):
        pltpu.async_copy(src3.at[f, pl.ds(0, 8), pl.ds(i0, cw)],
                         buf.at[pl.ds(0, 8), pl.ds(0, cw)], sem)
        pltpu.async_copy(src3.at[f, pl.ds(8, 8), pl.ds(i0, cw)],
                         buf.at[pl.ds(8, 8), pl.ds(0, cw)], sem)

    def b_drain_in(src3, buf, i0, cw, sem):
        pltpu.make_async_copy(src3.at[0, pl.ds(0, 8), pl.ds(i0, cw)],
                              buf.at[pl.ds(0, 8), pl.ds(0, cw)],
                              sem).wait()
        pltpu.make_async_copy(src3.at[0, pl.ds(8, 8), pl.ds(i0, cw)],
                              buf.at[pl.ds(8, 8), pl.ds(0, cw)],
                              sem).wait()

    def b_rows(buf, stg2, nc):
        def jrow(j, carry):
            def cbody(c, cc):
                o = pl.multiple_of(c * 16, 16)
                stg2[pl.ds(pl.multiple_of(j * _CW + o, 16), 16)] = \
                    buf[j, pl.ds(o, 16)]
                return cc
            lax.fori_loop(0, nc, cbody, 0)
            return carry
        lax.fori_loop(0, 16, jrow, 0)

    def b_out(out1d, f, c, stg2, sem):
        off = pl.multiple_of((f * _NCP + c) * _CB, 8)
        pltpu.async_copy(stg2, out1d.at[pl.ds(off, _CB)], sem)

    def b_drain_out(stg2, sem):
        pltpu.make_async_copy(utL.at[pl.ds(0, _CB)], stg2, sem).wait()

    def b_phase(src3, out1d, nplanes):
        ntask = nplanes * _NCH

        def task_in(c, buf, insem):
            @pl.when(c < ntask)
            def _():
                f = c // _NCH
                i0 = pl.multiple_of((c - f * _NCH) * _CW, 128)
                b_issue_in(src3, f, i0, _CW, buf, insem)

        def b_proc(c, t, buf, stg2, insem, osem):
            @pl.when(c < ntask)
            def _():
                b_drain_in(src3, buf, 0, _CW, insem)

                @pl.when(t > 0)
                def _():
                    b_drain_out(stg2, osem)
                b_rows(buf, stg2, _CW // 16)
                f = c // _NCH
                b_out(out1d, f, c - f * _NCH, stg2, osem)

        task_in(wid, colvA, semA)

        def loop(t, carry):
            cA = wid + 2 * t * _NW
            cB = wid + (2 * t + 1) * _NW
            task_in(cB, colvB, semB)
            b_proc(cA, t, colvA, stg2A, semA, semC)
            task_in(cA + 2 * _NW, colvA, semA)
            b_proc(cB, t, colvB, stg2B, semB, semD)
            return carry
        lax.fori_loop(0, ntask // (2 * _NW) + 1, loop, 0)

        @pl.when(wid < ntask)
        def _():
            b_drain_out(stg2A, semC)

        @pl.when(wid + _NW < ntask)
        def _():
            b_drain_out(stg2B, semD)

    b_phase(utT, utL, _NU)
    b_phase(irT, irL, _NI - 1)

    def b_tail(src3, out1d, f):
        b_issue_in(src3, f, _NCH * _CW, _TW, tailv, semA)
        b_drain_in(src3, tailv, _NCH * _CW, _TW, semA)
        b_rows(tailv, stg2A, _TW // 16)
        b_out(out1d, f, _NCH, stg2A, semA)
        b_drain_out(stg2A, semA)

    @pl.when(wid < _NU)
    def _():
        b_tail(utT, utL, wid)

    @pl.when(jnp.logical_and(wid >= _NU, wid < _NU + _NI - 1))
    def _():
        b_tail(irT, irL, wid - _NU)


def _sc_body(ucat, ccat, icat, histf, hmaskf, unum, inum, wun, bun2, win,
             bin2, utf, ctf, it0, irf, x_out, y_out,
             idxA, idxB, idxTA, idxTB, colgA, colgB, embA, embB,
             ids_all, hmask_all, hrA, hrB,
             pool_v, num_v, un_v, in_v, wun_v, bun_v, win_v, bin_v,
             semA, semB):
    info = plsc.get_sparse_core_info()
    wid = lax.axis_index("s") * info.num_cores + lax.axis_index("c")
    b0 = wid * _RPT
    bs = pl.ds(b0, _RPT)

    pltpu.sync_copy(wun, wun_v)
    pltpu.sync_copy(bun2, bun_v)
    pltpu.sync_copy(win, win_v)
    pltpu.sync_copy(bin2, bin_v)
    pltpu.sync_copy(unum.at[pl.ds(b0 * _D, _RPT * _D)], un_v)
    pltpu.sync_copy(inum.at[pl.ds(b0 * _D, _RPT * _D)], in_v)

    def unum_body(r, carry):
        uvec = un_v[pl.ds(pl.multiple_of(r * _D, _D), _D)]
        acc = bun_v[...]
        for k in range(_UNUM):
            acc = acc + uvec[k] * wun_v[pl.ds(k * _D, _D)]
        num_v[r, :] = acc
        return carry
    lax.fori_loop(0, _RPT, unum_body, 0)
    pltpu.sync_copy(num_v, x_out.at[bs, pl.ds(26 * _D, _D)])

    def inum_body(r, carry):
        ivec = in_v[pl.ds(pl.multiple_of(r * _D, _D), _D)]
        acc = bin_v[...]
        for k in range(_INUM):
            acc = acc + ivec[k] * win_v[pl.ds(k * _D, _D)]
        num_v[r, :] = acc
        return carry
    lax.fori_loop(0, _RPT, inum_body, 0)
    pltpu.sync_copy(num_v, y_out.at[bs, pl.ds(_NI * _D, _D)])

    i16 = lax.iota(jnp.int32, 16)

    def wprep(srcarr, srcf, f, table, idx_v, idxT_v, colg_v, sem):
        pltpu.sync_copy(
            srcarr.at[pl.ds(pl.multiple_of(srcf * _B + b0, 8), _RPT)],
            idx_v)
        jv = i16 * _CW

        def rxf(c, carry):
            idv = idx_v[pl.ds(pl.multiple_of(c * 16, 16), 16)]
            bvec = ((idv >> 9) * _CB + (idv & 511)
                    + f * (_NCP * _CB))
            for m in range(16):
                r = c * 16 + m
                idxT_v[pl.ds(pl.multiple_of(r * 16, 16), 16)] = \
                    jv + bvec[m]
            return carry
        lax.fori_loop(0, _RPT // 16, rxf, 0)

        def jissue(j, carry):
            jb = pl.multiple_of(j * 128, 128)
            pltpu.async_copy(table.at[idxT_v.at[pl.ds(jb, 128)]],
                             colg_v.at[pl.ds(jb, 128)], sem)
            return carry
        lax.fori_loop(0, 16, jissue, 0)

    def wfinish(dstbuf, col, colg_v, emb_v, sem):
        def jdrain(j, carry):
            jb = pl.multiple_of(j * 128, 128)
            pltpu.make_async_copy(utf.at[pl.ds(0, 128)],
                                  colg_v.at[pl.ds(jb, 128)], sem).wait()
            return carry
        lax.fori_loop(0, 16, jdrain, 0)

        def rtrans(rb, carry):
            for m in range(16):
                r = rb * 16 + m
                emb_v[r, :] = colg_v[pl.ds(pl.multiple_of(r * 16, 16), 16)]
            return carry
        lax.fori_loop(0, _RPT // 16, rtrans, 0)
        pltpu.sync_copy(emb_v,
                        dstbuf.at[bs, pl.ds(pl.multiple_of(col, 16), _D)])

    def ugrp(t, carry):
        f1 = 2 * t
        f2 = 2 * t + 1
        wprep(ucat, f1, f1, utf, idxA, idxTA, colgA, semA)

        @pl.when(f2 < _NU)
        def _():
            wprep(ucat, f2, f2, utf, idxB, idxTB, colgB, semB)
        wfinish(x_out, f1 * _D, colgA, embA, semA)

        @pl.when(f2 < _NU)
        def _():
            wfinish(x_out, f2 * _D, colgB, embB, semB)
        return carry
    lax.fori_loop(0, (_NU + 1) // 2, ugrp, 0)

    def igrp2(t, carry):
        f1 = 2 * t
        f2 = 2 * t + 1
        wprep(icat, f1 + 1, f1, irf, idxA, idxTA, colgA, semA)

        @pl.when(f2 < _NI - 1)
        def _():
            wprep(icat, f2 + 1, f2, irf, idxB, idxTB, colgB, semB)
        wfinish(y_out, (f1 + 1) * _D, colgA, embA, semA)

        @pl.when(f2 < _NI - 1)
        def _():
            wfinish(y_out, (f2 + 1) * _D, colgB, embB, semB)
        return carry
    lax.fori_loop(0, _NI // 2, igrp2, 0)

    for f in range(_NC):
        pltpu.sync_copy(ccat.at[pl.ds(f * _B + b0, _RPT)], idxA)
        if f:
            for c in range(_RPT // 16):
                sl = pl.ds(c * 16, 16)
                idxA[sl] = idxA[sl] + f * _VC
        pltpu.async_copy(ctf.at[idxA], embA, semA).wait()
        pltpu.sync_copy(embA, x_out.at[bs, pl.ds((_NU + f) * _D, _D)])

    pltpu.sync_copy(icat.at[pl.ds(b0, _RPT)], idxA)
    pltpu.async_copy(it0.at[idxA], embA, semA).wait()
    pltpu.sync_copy(embA, y_out.at[bs, pl.ds(0, _D)])

    pltpu.sync_copy(histf.at[pl.ds(b0 * _HL, _RPT * _HL)], ids_all)
    pltpu.sync_copy(hmaskf.at[pl.ds(b0 * _HL, _RPT * _HL)], hmask_all)

    def issue_stage(s, buf, sem):
        for k in range(_RPS):
            o = pl.multiple_of(s * _SID + k * _HL, 8)
            pltpu.async_copy(it0.at[ids_all.at[pl.ds(o, 128)]],
                             buf.at[pl.ds(k * _HL, 128)], sem)
            pltpu.async_copy(it0.at[ids_all.at[pl.ds(o + 128, _HL - 128)]],
                             buf.at[pl.ds(k * _HL + 128, _HL - 128)], sem)

    def drain_stage(buf, sem):
        for k in range(_RPS):
            pltpu.make_async_copy(it0.at[pl.ds(0, 128)],
                                  buf.at[pl.ds(k * _HL, 128)], sem).wait()
            pltpu.make_async_copy(it0.at[pl.ds(0, _HL - 128)],
                                  buf.at[pl.ds(k * _HL + 128, _HL - 128)],
                                  sem).wait()

    def compute_stage(s, buf):
        for k in range(_RPS):
            mbase = s * _SID + k * _HL
            zv = jnp.zeros((16,), jnp.float32)

            def acc_body(c, carry2):
                accs, ms = carry2
                accs = list(accs)
                mvec = hmask_all[pl.ds(pl.multiple_of(mbase + c * 16, 8), 16)]
                base = k * _HL + c * 16
                for j in range(16):
                    mj = mvec[j]
                    accs[j % 4] = accs[j % 4] + buf[base + j, :] * mj
                    ms = ms + mj
                return (tuple(accs), ms)
            accs, ms = lax.fori_loop(
                0, 12, acc_body, ((zv, zv, zv, zv), jnp.float32(0.0)))
            a0, a1, a2, a3 = accs
            mvec = hmask_all[pl.ds(pl.multiple_of(mbase + 192, 8), 16)]
            for j in range(8):
                mj = mvec[j]
                a0 = a0 + buf[k * _HL + 192 + j, :] * mj
                ms = ms + mj
            a = (a0 + a1) + (a2 + a3)
            pool_v[s * _RPS + k, :] = a / jnp.maximum(ms, 1e-6)

    issue_stage(0, hrA, semA)

    def hist_loop(t, carry):
        sA = 2 * t
        sB = 2 * t + 1
        issue_stage(sB, hrB, semB)
        drain_stage(hrA, semA)
        compute_stage(sA, hrA)
        issue_stage(lax.rem(sA + 2, _NST), hrA, semA)
        drain_stage(hrB, semB)
        compute_stage(sB, hrB)
        return carry
    lax.fori_loop(0, _NST // 2, hist_loop, 0)
    drain_stage(hrA, semA)

    pltpu.sync_copy(pool_v, x_out.at[bs, pl.ds(27 * _D, _D)])


def _tc_body(x_ref, y_ref, wu1, bu1, wu2, bu2, wi1, bi1, wi2, bi2,
             u_ref, i_ref):
    f32 = jnp.float32
    xb = x_ref[...]
    h = jnp.maximum(
        jnp.dot(xb, wu1[...], preferred_element_type=f32) + bu1[...], 0.0)
    uu = jnp.dot(h, wu2[...], preferred_element_type=f32) + bu2[...]
    n = jnp.sqrt(jnp.sum(uu * uu, axis=-1, keepdims=True))
    u_ref[...] = uu / jnp.maximum(n, 1e-12)

    yb = y_ref[...]
    h2 = jnp.maximum(
        jnp.dot(yb, wi1[...], preferred_element_type=f32) + bi1[...], 0.0)
    ii = jnp.dot(h2, wi2[...], preferred_element_type=f32) + bi2[...]
    n2 = jnp.sqrt(jnp.sum(ii * ii, axis=-1, keepdims=True))
    i_ref[...] = ii / jnp.maximum(n2, 1e-12)


def kernel(user_cat, user_num, ctx_cat, hist_ids, hist_mask, item_cat,
           item_num, user_tables, ctx_tables, item_table0, item_tables_rest,
           Wun, bun, Win, bin, Wu1, bu1, Wu2, bu2, Wi1, bi1, Wi2, bi2):
    f32 = jnp.float32
    ucat_f = user_cat.T.astype(jnp.int32).reshape(-1)
    ccat_f = ctx_cat.T.astype(jnp.int32).reshape(-1)
    icat_f = item_cat.T.astype(jnp.int32).reshape(-1)
    hist_flat = hist_ids.reshape(-1).astype(jnp.int32)
    hmask_flat = hist_mask.reshape(-1)
    unum_pad = jnp.pad(user_num, ((0, 0), (0, _D - _UNUM))).reshape(-1)
    inum_pad = jnp.pad(item_num, ((0, 0), (0, _D - _INUM))).reshape(-1)

    utT = jnp.swapaxes(user_tables, 1, 2)
    irT = jnp.swapaxes(item_tables_rest, 1, 2)

    mesh = plsc.VectorSubcoreMesh(core_axis_name="c", subcore_axis_name="s")
    repack = functools.partial(
        pl.kernel,
        mesh=mesh,
        out_type=[jax.ShapeDtypeStruct((_VI0 * _D,), f32),
                  jax.ShapeDtypeStruct((_NU * _NCP * _CB,), f32),
                  jax.ShapeDtypeStruct(((_NI - 1) * _NCP * _CB,), f32)],
        scratch_types=[
            pltpu.VMEM((128, _D), f32),
            pltpu.VMEM((128, _D), f32),
            pltpu.VMEM((128, _D), f32),
            pltpu.VMEM((128, _D), f32),
            pltpu.VMEM((16, _CW), f32),
            pltpu.VMEM((16, _CW), f32),
            pltpu.VMEM((16, _TW), f32),
            pltpu.VMEM((2048,), f32),
            pltpu.VMEM((2048,), f32),
            pltpu.VMEM((2048,), f32),
            pltpu.VMEM((2048,), f32),
            pltpu.VMEM((_CB,), f32),
            pltpu.VMEM((_CB,), f32),
            pltpu.SemaphoreType.DMA,
            pltpu.SemaphoreType.DMA,
            pltpu.SemaphoreType.DMA,
            pltpu.SemaphoreType.DMA,
            pltpu.SemaphoreType.DMA,
            pltpu.SemaphoreType.DMA,
            pltpu.SemaphoreType.DMA,
            pltpu.SemaphoreType.DMA,
        ],
    )(_repack_body)
    it0L, utL, irL = repack(item_table0, utT, irT)
    ct_flat = ctx_tables.reshape(_NC * _VC, _D)

    it0_lin = it0L.reshape(_VI0, _D)

    sc = functools.partial(
        pl.kernel,
        mesh=mesh,
        compiler_params=pltpu.CompilerParams(use_tc_tiling_on_sc=False),
        out_type=[jax.ShapeDtypeStruct((_B, _UIN), f32),
                  jax.ShapeDtypeStruct((_B, _IIN), f32)],
        scratch_types=[
            pltpu.VMEM((_RPT,), jnp.int32),
            pltpu.VMEM((_RPT,), jnp.int32),
            pltpu.VMEM((16 * _RPT,), jnp.int32),
            pltpu.VMEM((16 * _RPT,), jnp.int32),
            pltpu.VMEM((16 * _RPT,), f32),
            pltpu.VMEM((16 * _RPT,), f32),
            pltpu.VMEM((_RPT, _D), f32),
            pltpu.VMEM((_RPT, _D), f32),
            pltpu.VMEM((_RPT * _HL,), jnp.int32),
            pltpu.VMEM((_RPT * _HL,), f32),
            pltpu.VMEM((_SID, _D), f32),
            pltpu.VMEM((_SID, _D), f32),
            pltpu.VMEM((_RPT, _D), f32),
            pltpu.VMEM((_RPT, _D), f32),
            pltpu.VMEM((_RPT * _D,), f32),
            pltpu.VMEM((_RPT * _D,), f32),
            pltpu.VMEM((_UNUM * _D,), f32),
            pltpu.VMEM((_D,), f32),
            pltpu.VMEM((_INUM * _D,), f32),
            pltpu.VMEM((_D,), f32),
            pltpu.SemaphoreType.DMA,
            pltpu.SemaphoreType.DMA,
        ],
    )(_sc_body)
    x, y = sc(ucat_f, ccat_f, icat_f, hist_flat, hmask_flat, unum_pad,
              inum_pad, Wun.reshape(-1), bun, Win.reshape(-1), bin,
              utL, ct_flat, it0_lin, irL)

    bm = 1024
    grid = _B // bm
    full = lambda i: (0, 0)
    u, i = pl.pallas_call(
        _tc_body,
        grid=(grid,),
        in_specs=[
            pl.BlockSpec((bm, _UIN), lambda i: (i, 0)),
            pl.BlockSpec((bm, _IIN), lambda i: (i, 0)),
            pl.BlockSpec((_UIN, _HID), full),
            pl.BlockSpec((1, _HID), full),
            pl.BlockSpec((_HID, _TOW), full),
            pl.BlockSpec((1, _TOW), full),
            pl.BlockSpec((_IIN, _HID), full),
            pl.BlockSpec((1, _HID), full),
            pl.BlockSpec((_HID, _TOW), full),
            pl.BlockSpec((1, _TOW), full),
        ],
        out_specs=[pl.BlockSpec((bm, _TOW), lambda i: (i, 0)),
                   pl.BlockSpec((bm, _TOW), lambda i: (i, 0))],
        out_shape=[jax.ShapeDtypeStruct((_B, _TOW), f32),
                   jax.ShapeDtypeStruct((_B, _TOW), f32)],
    )(x, y, Wu1, bu1.reshape(1, _HID), Wu2, bu2.reshape(1, _TOW),
      Wi1, bi1.reshape(1, _HID), Wi2, bi2.reshape(1, _TOW))
    return (u, i)


# speedup vs baseline: 1.5674x; 1.0003x over previous
import functools

import jax
import jax.numpy as jnp
from jax import lax
from jax.experimental import pallas as pl
from jax.experimental.pallas import tpu as pltpu
from jax.experimental.pallas import tpu_sc as plsc

_B = 4096
_D = 16
_NU = 23
_NC = 3
_NI = 8
_HL = 200
_VU = 100000
_VC = 1000
_VI0 = 1000000
_VIR = 100000
_UNUM = 4
_INUM = 6
_HID = 128
_TOW = 64
_UIN = _NU * _D + _NC * _D + 2 * _D
_IIN = _NI * _D + _D

_NW = 32
_RPT = _B // _NW
_RPS = 4
_NST = _RPT // _NW * _NW // _RPS * 1
_NST = _RPT // _RPS
_SID = _RPS * _HL

_CW = 512
_NCH = _VU // _CW
_TW = _VU - _NCH * _CW
_NCP = _NCH + 1
_CB = 16 * _CW


def _repack_body(it0, utT, irT, it0L, utL, irL,
                 srcv0, srcv1, srcv2, srcv3, colvA, colvB, tailv,
                 stg0, stg1, stg2, stg3, stg2A, stg2B,
                 semA, semB, semC, semD, osem0, osem1, osem2, osem3):
    info = plsc.get_sparse_core_info()
    wid = lax.axis_index("s") * info.num_cores + lax.axis_index("c")
    srcvs = [srcv0, srcv1, srcv2, srcv3]
    stgs = [stg0, stg1, stg2, stg3]
    isems = [semA, semB, semC, semD]
    osems = [osem0, osem1, osem2, osem3]

    nfull = _VI0 // 128

    def a_issue_in(c, srcv, sem):
        @pl.when(c < nfull)
        def _():
            r0 = pl.multiple_of(c * 128, 8)
            pltpu.async_copy(it0.at[pl.ds(r0, 128), :], srcv, sem)

    def a_drain_in(srcv, sem):
        pltpu.make_async_copy(it0.at[pl.ds(0, 128), :], srcv, sem).wait()

    def a_proc(c, t, srcv, stg, sem, osem):
        @pl.when(c < nfull)
        def _():
            a_drain_in(srcv, sem)

            @pl.when(t > 0)
            def _():
                pltpu.make_async_copy(it0L.at[pl.ds(0, 2048)], stg,
                                      osem).wait()

            def rbody(r, cc):
                stg[pl.ds(pl.multiple_of(r * 16, 16), 16)] = srcv[r, :]
                return cc
            lax.fori_loop(0, 128, rbody, 0)
            pltpu.async_copy(stg, it0L.at[pl.ds(c * 2048, 2048)], osem)

    for k in range(4):
        a_issue_in(wid + k * _NW, srcvs[k], isems[k])

    def it0_loop(t, carry):
        for k in range(4):
            c = wid + (4 * t + k) * _NW
            a_proc(c, t, srcvs[k], stgs[k], isems[k], osems[k])
            a_issue_in(c + 4 * _NW, srcvs[k], isems[k])
        return carry
    lax.fori_loop(0, nfull // (4 * _NW) + 1, it0_loop, 0)
    for k in range(4):
        @pl.when(wid + k * _NW < nfull)
        def _(k=k):
            pltpu.make_async_copy(it0L.at[pl.ds(0, 2048)], stgs[k],
                                  osems[k]).wait()

    @pl.when(wid == 0)
    def _():
        r0 = nfull * 128
        pltpu.sync_copy(it0.at[pl.ds(r0, 64), :],
                        srcv0.at[pl.ds(0, 64), :])

        def rbody(r, cc):
            stg0[pl.ds(pl.multiple_of(r * 16, 16), 16)] = srcv0[r, :]
            return cc
        lax.fori_loop(0, 64, rbody, 0)
        pltpu.sync_copy(stg0.at[pl.ds(0, 1024)],
                        it0L.at[pl.ds(r0 * 16, 1024)])

    def b_issue_in(src3, f, i0, cw, buf, sem):
        pltpu.async_copy(src3.at[f, pl.ds(0, 8), pl.ds(i0, cw)],
                         buf.at[pl.ds(0, 8), pl.ds(0, cw)], sem)
        pltpu.async_copy(src3.at[f, pl.ds(8, 8), pl.ds(i0, cw)],
                         buf.at[pl.ds(8, 8), pl.ds(0, cw)], sem)

    def b_drain_in(src3, buf, i0, cw, sem):
        pltpu.make_async_copy(src3.at[0, pl.ds(0, 8), pl.ds(i0, cw)],
                              buf.at[pl.ds(0, 8), pl.ds(0, cw)],
                              sem).wait()
        pltpu.make_async_copy(src3.at[0, pl.ds(8, 8), pl.ds(i0, cw)],
                              buf.at[pl.ds(8, 8), pl.ds(0, cw)],
                              sem).wait()

    def b_rows(buf, stg2, nc):
        def jrow(j, carry):
            def cbody(c, cc):
                o = pl.multiple_of(c * 16, 16)
                stg2[pl.ds(pl.multiple_of(j * _CW + o, 16), 16)] = \
                    buf[j, pl.ds(o, 16)]
                return cc
            lax.fori_loop(0, nc, cbody, 0)
            return carry
        lax.fori_loop(0, 16, jrow, 0)

    def b_out(out1d, f, c, stg2, sem):
        off = pl.multiple_of((f * _NCP + c) * _CB, 8)
        pltpu.async_copy(stg2, out1d.at[pl.ds(off, _CB)], sem)

    def b_drain_out(stg2, sem):
        pltpu.make_async_copy(utL.at[pl.ds(0, _CB)], stg2, sem).wait()

    def b_phase(src3, out1d, nplanes):
        ntask = nplanes * _NCH

        def task_in(c, buf, insem):
            @pl.when(c < ntask)
            def _():
                f = c // _NCH
                i0 = pl.multiple_of((c - f * _NCH) * _CW, 128)
                b_issue_in(src3, f, i0, _CW, buf, insem)

        def b_proc(c, t, buf, stg2, insem, osem):
            @pl.when(c < ntask)
            def _():
                b_drain_in(src3, buf, 0, _CW, insem)

                @pl.when(t > 0)
                def _():
                    b_drain_out(stg2, osem)
                b_rows(buf, stg2, _CW // 16)
                f = c // _NCH
                b_out(out1d, f, c - f * _NCH, stg2, osem)

        task_in(wid, colvA, semA)

        def loop(t, carry):
            cA = wid + 2 * t * _NW
            cB = wid + (2 * t + 1) * _NW
            task_in(cB, colvB, semB)
            b_proc(cA, t, colvA, stg2A, semA, semC)
            task_in(cA + 2 * _NW, colvA, semA)
            b_proc(cB, t, colvB, stg2B, semB, semD)
            return carry
        lax.fori_loop(0, ntask // (2 * _NW) + 1, loop, 0)

        @pl.when(wid < ntask)
        def _():
            b_drain_out(stg2A, semC)

        @pl.when(wid + _NW < ntask)
        def _():
            b_drain_out(stg2B, semD)

    b_phase(utT, utL, _NU)
    b_phase(irT, irL, _NI - 1)

    def b_tail(src3, out1d, f):
        b_issue_in(src3, f, _NCH * _CW, _TW, tailv, semA)
        b_drain_in(src3, tailv, _NCH * _CW, _TW, semA)
        b_rows(tailv, stg2A, _TW // 16)
        b_out(out1d, f, _NCH, stg2A, semA)
        b_drain_out(stg2A, semA)

    @pl.when(wid < _NU)
    def _():
        b_tail(utT, utL, wid)

    @pl.when(jnp.logical_and(wid >= _NU, wid < _NU + _NI - 1))
    def _():
        b_tail(irT, irL, wid - _NU)


def _sc_body(ucat, ccat, icat, histf, hmaskf, unum, inum, wun, bun2, win,
             bin2, utf, ctf, it0, irf, x_out, y_out,
             idxA, idxB, idxTA, idxTB, colgA, colgB, embA, embB,
             ids_all, hmask_all, hrA, hrB,
             pool_v, num_v, un_v, in_v, wun_v, bun_v, win_v, bin_v,
             semA, semB):
    info = plsc.get_sparse_core_info()
    wid = lax.axis_index("s") * info.num_cores + lax.axis_index("c")
    b0 = wid * _RPT
    bs = pl.ds(b0, _RPT)

    pltpu.sync_copy(wun, wun_v)
    pltpu.sync_copy(bun2, bun_v)
    pltpu.sync_copy(win, win_v)
    pltpu.sync_copy(bin2, bin_v)
    pltpu.sync_copy(unum.at[pl.ds(b0 * _D, _RPT * _D)], un_v)
    pltpu.sync_copy(inum.at[pl.ds(b0 * _D, _RPT * _D)], in_v)

    def unum_body(r, carry):
        uvec = un_v[pl.ds(pl.multiple_of(r * _D, _D), _D)]
        acc = bun_v[...]
        for k in range(_UNUM):
            acc = acc + uvec[k] * wun_v[pl.ds(k * _D, _D)]
        num_v[r, :] = acc
        return carry
    lax.fori_loop(0, _RPT, unum_body, 0)
    pltpu.sync_copy(num_v, x_out.at[bs, pl.ds(26 * _D, _D)])

    def inum_body(r, carry):
        ivec = in_v[pl.ds(pl.multiple_of(r * _D, _D), _D)]
        acc = bin_v[...]
        for k in range(_INUM):
            acc = acc + ivec[k] * win_v[pl.ds(k * _D, _D)]
        num_v[r, :] = acc
        return carry
    lax.fori_loop(0, _RPT, inum_body, 0)
    pltpu.sync_copy(num_v, y_out.at[bs, pl.ds(_NI * _D, _D)])

    i16 = lax.iota(jnp.int32, 16)

    def wprep(srcarr, srcf, f, table, idx_v, idxT_v, colg_v, sem):
        pltpu.sync_copy(
            srcarr.at[pl.ds(pl.multiple_of(srcf * _B + b0, 8), _RPT)],
            idx_v)
        jv = i16 * _CW

        def rxf(c, carry):
            idv = idx_v[pl.ds(pl.multiple_of(c * 16, 16), 16)]
            bvec = ((idv >> 9) * _CB + (idv & 511)
                    + f * (_NCP * _CB))
            for m in range(16):
                r = c * 16 + m
                idxT_v[pl.ds(pl.multiple_of(r * 16, 16), 16)] = \
                    jv + bvec[m]
            return carry
        lax.fori_loop(0, _RPT // 16, rxf, 0)

        def jissue(j, carry):
            jb = pl.multiple_of(j * 128, 128)
            pltpu.async_copy(table.at[idxT_v.at[pl.ds(jb, 128)]],
                             colg_v.at[pl.ds(jb, 128)], sem)
            return carry
        lax.fori_loop(0, 16, jissue, 0)

    def wfinish(dstbuf, col, colg_v, emb_v, sem):
        def jdrain(j, carry):
            jb = pl.multiple_of(j * 128, 128)
            pltpu.make_async_copy(utf.at[pl.ds(0, 128)],
                                  colg_v.at[pl.ds(jb, 128)], sem).wait()
            return carry
        lax.fori_loop(0, 16, jdrain, 0)

        def rtrans(rb, carry):
            for m in range(16):
                r = rb * 16 + m
                emb_v[r, :] = colg_v[pl.ds(pl.multiple_of(r * 16, 16), 16)]
            return carry
        lax.fori_loop(0, _RPT // 16, rtrans, 0)
        pltpu.sync_copy(emb_v,
                        dstbuf.at[bs, pl.ds(pl.multiple_of(col, 16), _D)])

    def ugrp(t, carry):
        f1 = 2 * t
        f2 = 2 * t + 1
        wprep(ucat, f1, f1, utf, idxA, idxTA, colgA, semA)

        @pl.when(f2 < _NU)
        def _():
            wprep(ucat, f2, f2, utf, idxB, idxTB, colgB, semB)
        wfinish(x_out, f1 * _D, colgA, embA, semA)

        @pl.when(f2 < _NU)
        def _():
            wfinish(x_out, f2 * _D, colgB, embB, semB)
        return carry
    lax.fori_loop(0, (_NU + 1) // 2, ugrp, 0)

    def igrp2(t, carry):
        f1 = 2 * t
        f2 = 2 * t + 1
        wprep(icat, f1 + 1, f1, irf, idxA, idxTA, colgA, semA)

        @pl.when(f2 < _NI - 1)
        def _():
            wprep(icat, f2 + 1, f2, irf, idxB, idxTB, colgB, semB)
        wfinish(y_out, (f1 + 1) * _D, colgA, embA, semA)

        @pl.when(f2 < _NI - 1)
        def _():
            wfinish(y_out, (f2 + 1) * _D, colgB, embB, semB)
        return carry
    lax.fori_loop(0, _NI // 2, igrp2, 0)

    for f in range(_NC):
        pltpu.sync_copy(ccat.at[pl.ds(f * _B + b0, _RPT)], idxA)
        if f:
            for c in range(_RPT // 16):
                sl = pl.ds(c * 16, 16)
                idxA[sl] = idxA[sl] + f * _VC
        pltpu.async_copy(ctf.at[idxA], embA, semA).wait()
        pltpu.sync_copy(embA, x_out.at[bs, pl.ds((_NU + f) * _D, _D)])

    pltpu.sync_copy(icat.at[pl.ds(b0, _RPT)], idxA)
    pltpu.async_copy(it0.at[idxA], embA, semA).wait()
    pltpu.sync_copy(embA, y_out.at[bs, pl.ds(0, _D)])

    pltpu.sync_copy(histf.at[pl.ds(b0 * _HL, _RPT * _HL)], ids_all)
    pltpu.sync_copy(hmaskf.at[pl.ds(b0 * _HL, _RPT * _HL)], hmask_all)

    def issue_stage(s, buf, sem):
        for k in range(_RPS):
            o = pl.multiple_of(s * _SID + k * _HL, 8)
            pltpu.async_copy(it0.at[ids_all.at[pl.ds(o, 128)]],
                             buf.at[pl.ds(k * _HL, 128)], sem)
            pltpu.async_copy(it0.at[ids_all.at[pl.ds(o + 128, _HL - 128)]],
                             buf.at[pl.ds(k * _HL + 128, _HL - 128)], sem)

    def drain_stage(buf, sem):
        for k in range(_RPS):
            pltpu.make_async_copy(it0.at[pl.ds(0, 128)],
                                  buf.at[pl.ds(k * _HL, 128)], sem).wait()
            pltpu.make_async_copy(it0.at[pl.ds(0, _HL - 128)],
                                  buf.at[pl.ds(k * _HL + 128, _HL - 128)],
                                  sem).wait()

    def compute_stage(s, buf):
        for k in range(_RPS):
            mbase = s * _SID + k * _HL
            zv = jnp.zeros((16,), jnp.float32)

            def acc_body(c, carry2):
                accs, ms = carry2
                accs = list(accs)
                mvec = hmask_all[pl.ds(pl.multiple_of(mbase + c * 16, 8), 16)]
                base = k * _HL + c * 16
                for j in range(16):
                    mj = mvec[j]
                    accs[j % 4] = accs[j % 4] + buf[base + j, :] * mj
                    ms = ms + mj
                return (tuple(accs), ms)
            accs, ms = lax.fori_loop(
                0, 12, acc_body, ((zv, zv, zv, zv), jnp.float32(0.0)))
            a0, a1, a2, a3 = accs
            mvec = hmask_all[pl.ds(pl.multiple_of(mbase + 192, 8), 16)]
            for j in range(8):
                mj = mvec[j]
                a0 = a0 + buf[k * _HL + 192 + j, :] * mj
                ms = ms + mj
            a = (a0 + a1) + (a2 + a3)
            pool_v[s * _RPS + k, :] = a / jnp.maximum(ms, 1e-6)

    issue_stage(0, hrA, semA)

    def hist_loop(t, carry):
        sA = 2 * t
        sB = 2 * t + 1
        issue_stage(sB, hrB, semB)
        drain_stage(hrA, semA)
        compute_stage(sA, hrA)
        issue_stage(lax.rem(sA + 2, _NST), hrA, semA)
        drain_stage(hrB, semB)
        compute_stage(sB, hrB)
        return carry
    lax.fori_loop(0, _NST // 2, hist_loop, 0)
    drain_stage(hrA, semA)

    pltpu.sync_copy(pool_v, x_out.at[bs, pl.ds(27 * _D, _D)])


def _tc_body(x_ref, y_ref, wu1, bu1, wu2, bu2, wi1, bi1, wi2, bi2,
             u_ref, i_ref):
    f32 = jnp.float32
    xb = x_ref[...]
    h = jnp.maximum(
        jnp.dot(xb, wu1[...], preferred_element_type=f32) + bu1[...], 0.0)
    uu = jnp.dot(h, wu2[...], preferred_element_type=f32) + bu2[...]
    n = jnp.sqrt(jnp.sum(uu * uu, axis=-1, keepdims=True))
    u_ref[...] = uu / jnp.maximum(n, 1e-12)

    yb = y_ref[...]
    h2 = jnp.maximum(
        jnp.dot(yb, wi1[...], preferred_element_type=f32) + bi1[...], 0.0)
    ii = jnp.dot(h2, wi2[...], preferred_element_type=f32) + bi2[...]
    n2 = jnp.sqrt(jnp.sum(ii * ii, axis=-1, keepdims=True))
    i_ref[...] = ii / jnp.maximum(n2, 1e-12)


def kernel(user_cat, user_num, ctx_cat, hist_ids, hist_mask, item_cat,
           item_num, user_tables, ctx_tables, item_table0, item_tables_rest,
           Wun, bun, Win, bin, Wu1, bu1, Wu2, bu2, Wi1, bi1, Wi2, bi2):
    f32 = jnp.float32
    ucat_f = user_cat.T.astype(jnp.int32).reshape(-1)
    ccat_f = ctx_cat.T.astype(jnp.int32).reshape(-1)
    icat_f = item_cat.T.astype(jnp.int32).reshape(-1)
    unum_pad = jnp.pad(user_num, ((0, 0), (0, _D - _UNUM))).reshape(-1)
    inum_pad = jnp.pad(item_num, ((0, 0), (0, _D - _INUM))).reshape(-1)

    utT = jnp.swapaxes(user_tables, 1, 2)
    irT = jnp.swapaxes(item_tables_rest, 1, 2)

    mesh = plsc.VectorSubcoreMesh(core_axis_name="c", subcore_axis_name="s")
    repack = functools.partial(
        pl.kernel,
        mesh=mesh,
        out_type=[jax.ShapeDtypeStruct((_VI0 * _D,), f32),
                  jax.ShapeDtypeStruct((_NU * _NCP * _CB,), f32),
                  jax.ShapeDtypeStruct(((_NI - 1) * _NCP * _CB,), f32)],
        scratch_types=[
            pltpu.VMEM((128, _D), f32),
            pltpu.VMEM((128, _D), f32),
            pltpu.VMEM((128, _D), f32),
            pltpu.VMEM((128, _D), f32),
            pltpu.VMEM((16, _CW), f32),
            pltpu.VMEM((16, _CW), f32),
            pltpu.VMEM((16, _TW), f32),
            pltpu.VMEM((2048,), f32),
            pltpu.VMEM((2048,), f32),
            pltpu.VMEM((2048,), f32),
            pltpu.VMEM((2048,), f32),
            pltpu.VMEM((_CB,), f32),
            pltpu.VMEM((_CB,), f32),
            pltpu.SemaphoreType.DMA,
            pltpu.SemaphoreType.DMA,
            pltpu.SemaphoreType.DMA,
            pltpu.SemaphoreType.DMA,
            pltpu.SemaphoreType.DMA,
            pltpu.SemaphoreType.DMA,
            pltpu.SemaphoreType.DMA,
            pltpu.SemaphoreType.DMA,
        ],
    )(_repack_body)
    it0L, utL, irL = repack(item_table0, utT, irT)
    hist_flat = hist_ids.reshape(-1).astype(jnp.int32)
    hmask_flat = hist_mask.reshape(-1)
    ct_flat = ctx_tables.reshape(_NC * _VC, _D)

    it0_lin = it0L.reshape(_VI0, _D)

    sc = functools.partial(
        pl.kernel,
        mesh=mesh,
        compiler_params=pltpu.CompilerParams(use_tc_tiling_on_sc=False),
        out_type=[jax.ShapeDtypeStruct((_B, _UIN), f32),
                  jax.ShapeDtypeStruct((_B, _IIN), f32)],
        scratch_types=[
            pltpu.VMEM((_RPT,), jnp.int32),
            pltpu.VMEM((_RPT,), jnp.int32),
            pltpu.VMEM((16 * _RPT,), jnp.int32),
            pltpu.VMEM((16 * _RPT,), jnp.int32),
            pltpu.VMEM((16 * _RPT,), f32),
            pltpu.VMEM((16 * _RPT,), f32),
            pltpu.VMEM((_RPT, _D), f32),
            pltpu.VMEM((_RPT, _D), f32),
            pltpu.VMEM((_RPT * _HL,), jnp.int32),
            pltpu.VMEM((_RPT * _HL,), f32),
            pltpu.VMEM((_SID, _D), f32),
            pltpu.VMEM((_SID, _D), f32),
            pltpu.VMEM((_RPT, _D), f32),
            pltpu.VMEM((_RPT, _D), f32),
            pltpu.VMEM((_RPT * _D,), f32),
            pltpu.VMEM((_RPT * _D,), f32),
            pltpu.VMEM((_UNUM * _D,), f32),
            pltpu.VMEM((_D,), f32),
            pltpu.VMEM((_INUM * _D,), f32),
            pltpu.VMEM((_D,), f32),
            pltpu.SemaphoreType.DMA,
            pltpu.SemaphoreType.DMA,
        ],
    )(_sc_body)
    x, y = sc(ucat_f, ccat_f, icat_f, hist_flat, hmask_flat, unum_pad,
              inum_pad, Wun.reshape(-1), bun, Win.reshape(-1), bin,
              utL, ct_flat, it0_lin, irL)

    bm = 1024
    grid = _B // bm
    full = lambda i: (0, 0)
    u, i = pl.pallas_call(
        _tc_body,
        grid=(grid,),
        in_specs=[
            pl.BlockSpec((bm, _UIN), lambda i: (i, 0)),
            pl.BlockSpec((bm, _IIN), lambda i: (i, 0)),
            pl.BlockSpec((_UIN, _HID), full),
            pl.BlockSpec((1, _HID), full),
            pl.BlockSpec((_HID, _TOW), full),
            pl.BlockSpec((1, _TOW), full),
            pl.BlockSpec((_IIN, _HID), full),
            pl.BlockSpec((1, _HID), full),
            pl.BlockSpec((_HID, _TOW), full),
            pl.BlockSpec((1, _TOW), full),
        ],
        out_specs=[pl.BlockSpec((bm, _TOW), lambda i: (i, 0)),
                   pl.BlockSpec((bm, _TOW), lambda i: (i, 0))],
        out_shape=[jax.ShapeDtypeStruct((_B, _TOW), f32),
                   jax.ShapeDtypeStruct((_B, _TOW), f32)],
    )(x, y, Wu1, bu1.reshape(1, _HID), Wu2, bu2.reshape(1, _TOW),
      Wi1, bi1.reshape(1, _HID), Wi2, bi2.reshape(1, _TOW))
    return (u, i)
